# Initial kernel scaffold; baseline (speedup 1.0000x reference)
import jax
import jax.numpy as jnp
from jax.experimental import pallas as pl


def kernel(x, gate_w, gate_b, fc1_w, fc1_b, fc2_w, fc2_b):
    raise NotImplementedError("write your pallas kernel here")


# speedup vs baseline: 1.0046x; 1.0046x over previous
import functools

import jax
import jax.numpy as jnp
from jax import lax
from jax.experimental import pallas as pl
from jax.experimental.pallas import tpu as pltpu
from jax.experimental.pallas import tpu_sc as plsc

H = 1024
FF = (512, 768, 1024, 1536, 1536, 1024, 768, 512)
E = 8
T = 2048
TOPK = 2

BLK = 512
BF = 256
KF = tuple(f // BF for f in FF)
FSTARTB = (0, 2, 5, 9, 15, 21, 25, 28)
NFB = sum(KF)
NB_MAX = 15
SCRATCH_BLK = NB_MAX
R_MAX = (NB_MAX + 1) * BLK
W_MAX = 80
LANES = 128

NC = 2
NS = 16
NW = NC * NS
TPW = T // NW


def _k1_body(x_ref, gw_ref, gb_ref, const_ref, misc_ref, sched_ref):
    f32 = jnp.float32
    x = x_ref[...]
    gw = gw_ref[...]
    logits = lax.dot_general(x.astype(jnp.bfloat16), gw.astype(jnp.bfloat16),
                             (((1,), (1,)), ((), ())),
                             preferred_element_type=f32)
    logits = logits + gb_ref[...]
    lane = lax.broadcasted_iota(jnp.int32, (T, LANES), 1)

    m1 = jnp.max(logits, axis=1, keepdims=True)
    a1 = jnp.min(jnp.where(logits == m1, lane, LANES), axis=1, keepdims=True)
    l2 = jnp.where(lane == a1, -jnp.inf, logits)
    m2 = jnp.max(l2, axis=1, keepdims=True)
    a2 = jnp.min(jnp.where(l2 == m2, lane, LANES), axis=1, keepdims=True)
    e2 = jnp.exp(m2 - m1)
    w1 = 1.0 / (1.0 + e2)
    w2 = e2 / (1.0 + e2)

    m_ind = jnp.logical_or(lane == a1, lane == a2).astype(f32)
    r_iota = lax.broadcasted_iota(jnp.int32, (LANES, LANES), 0)
    c_iota = lax.broadcasted_iota(jnp.int32, (LANES, LANES), 1)
    tril_strict = (c_iota < r_iota).astype(f32)
    triu_strict = (r_iota < c_iota).astype(f32)
    acc = jnp.zeros((1, LANES), f32)
    ranks = []
    for i in range(T // LANES):
        mi = m_ind[i * LANES:(i + 1) * LANES, :]
        ranks.append(lax.dot_general(
            tril_strict, mi, (((1,), (0,)), ((), ())),
            preferred_element_type=f32,
            precision=lax.Precision.HIGHEST) + acc)
        acc = acc + jnp.sum(mi, axis=0, keepdims=True)
    rank = jnp.concatenate(ranks, axis=0)
    counts = acc

    nb = jnp.floor((counts + (BLK - 1)) * (1.0 / BLK))
    sb = lax.dot_general(nb, triu_strict, (((1,), (0,)), ((), ())),
                         preferred_element_type=f32,
                         precision=lax.Precision.HIGHEST)
    kfv = const_ref[0:1, :]
    work = nb * kfv
    ws = lax.dot_general(work, triu_strict, (((1,), (0,)), ((), ())),
                         preferred_element_type=f32,
                         precision=lax.Precision.HIGHEST)
    w_total = jnp.sum(work, axis=1, keepdims=True)

    dest = sb * float(BLK) + rank
    p1 = jnp.sum(jnp.where(lane == a1, dest, 0.0), axis=1, keepdims=True)
    p2 = jnp.sum(jnp.where(lane == a2, dest, 0.0), axis=1, keepdims=True)

    misc = jnp.concatenate(
        [p1, p2, w1, w2, a1.astype(f32), a2.astype(f32),
         jnp.zeros((T, LANES - 6), f32)], axis=1)
    misc_ref[...] = misc

    gi = lax.broadcasted_iota(jnp.int32, (LANES, LANES), 0).astype(f32)
    gcol = gi[:, 0:1]
    elane = lax.broadcasted_iota(jnp.int32, (LANES, LANES), 1)
    ws_b = jnp.broadcast_to(ws, (LANES, LANES))
    in_range = jnp.logical_and(gi >= ws_b, elane < E).astype(f32)
    e_g = jnp.sum(in_range, axis=1, keepdims=True) - 1.0
    onehot = (elane.astype(f32) == e_g).astype(f32)
    ws_g = jnp.sum(onehot * ws_b, axis=1, keepdims=True)
    kf_g = jnp.sum(onehot * kfv, axis=1, keepdims=True)
    sb_g = jnp.sum(onehot * jnp.broadcast_to(sb, (LANES, LANES)),
                   axis=1, keepdims=True)
    fsb = const_ref[1:2, :]
    fsb_g = jnp.sum(onehot * fsb, axis=1, keepdims=True)
    local = gcol - ws_g
    r_g = jnp.floor(local / jnp.maximum(kf_g, 1.0))
    j_g = local - r_g * kf_g
    valid = (gcol < w_total).astype(f32)
    row_blk = jnp.where(valid > 0, sb_g + r_g, float(SCRATCH_BLK))
    f_blk = jnp.where(valid > 0, fsb_g + j_g, 0.0)
    first = jnp.where(valid > 0, (j_g == 0.0).astype(f32), 0.0)
    last = jnp.where(valid > 0, (j_g == kf_g - 1.0).astype(f32), 0.0)
    eid = jnp.where(valid > 0, e_g, 0.0)
    sched_ref[...] = jnp.concatenate(
        [row_blk, f_blk, first, last, eid,
         jnp.zeros((LANES, LANES - 5), f32)], axis=1).astype(jnp.int32)


def _run_k1(x2d, gate_w, gate_b):
    gwp = jnp.zeros((LANES, H), jnp.float32).at[:E].set(gate_w)
    gbp = jnp.full((1, LANES), -1e30, jnp.float32).at[0, :E].set(gate_b)
    consts = jnp.zeros((2, LANES), jnp.float32)
    consts = consts.at[0, :E].set(jnp.asarray(KF, jnp.float32))
    consts = consts.at[1, :E].set(jnp.asarray(FSTARTB, jnp.float32))
    return pl.pallas_call(
        _k1_body,
        out_shape=(jax.ShapeDtypeStruct((T, LANES), jnp.float32),
                   jax.ShapeDtypeStruct((LANES, LANES), jnp.int32)),
    )(x2d, gwp, gbp, consts)


K2_CH = 32
K2_NCH = TPW // K2_CH


def _k2_body(x_hbm, pos0_hbm, pos1_hbm, xs_hbm, xbuf, idx, sem):
    wid = lax.axis_index("s") * NC + lax.axis_index("c")
    base = wid * TPW
    for c in range(K2_NCH):
        pltpu.sync_copy(pos0_hbm.at[pl.ds(base + c * K2_CH, K2_CH)],
                        idx.at[2 * c])
        pltpu.sync_copy(pos1_hbm.at[pl.ds(base + c * K2_CH, K2_CH)],
                        idx.at[2 * c + 1])
    for c in range(K2_NCH):
        pltpu.sync_copy(x_hbm.at[pl.ds(base + c * K2_CH, K2_CH), :], xbuf)
        cp0 = pltpu.async_copy(xbuf, xs_hbm.at[idx.at[2 * c]], sem)
        cp1 = pltpu.async_copy(xbuf, xs_hbm.at[idx.at[2 * c + 1]], sem)
        cp0.wait()
        cp1.wait()


def _run_k2(x2d, pos0, pos1):
    mesh = plsc.VectorSubcoreMesh(core_axis_name="c", subcore_axis_name="s")
    return pl.kernel(
        _k2_body,
        mesh=mesh,
        out_type=jax.ShapeDtypeStruct((R_MAX, H), jnp.float32),
        scratch_types=[
            pltpu.VMEM((K2_CH, H), jnp.float32),
            pltpu.VMEM((2 * K2_NCH, K2_CH), jnp.int32),
            pltpu.SemaphoreType.DMA,
        ],
    )(x2d, pos0, pos1)


def _k3_body(rb_ref, fb_ref, fi_ref, la_ref, ei_ref,
             x_ref, w1_ref, w2_ref, b1_ref, b2_ref, y_ref):
    g = pl.program_id(0)
    valid = rb_ref[g] != SCRATCH_BLK
    first = fi_ref[g] == 1
    last = la_ref[g] == 1

    @pl.when(valid)
    def _():
        xb = x_ref[...].astype(jnp.bfloat16)
        w1 = w1_ref[...].astype(jnp.bfloat16)
        pre = lax.dot_general(xb, w1, (((1,), (1,)), ((), ())),
                              preferred_element_type=jnp.float32)
        pre = pre + b1_ref[0]
        h = 0.5 * pre * (1.0 + lax.erf(pre * 0.7071067811865475))
        hb = h.astype(jnp.bfloat16)
        w2 = w2_ref[...].astype(jnp.bfloat16)
        y = lax.dot_general(hb, w2, (((1,), (0,)), ((), ())),
                            preferred_element_type=jnp.float32)

        @pl.when(first)
        def _():
            y_ref[...] = y

        @pl.when(jnp.logical_not(first))
        def _():
            y_ref[...] = y_ref[...] + y

        @pl.when(last)
        def _():
            y_ref[...] = y_ref[...] + b2_ref[0]


def _run_k3(xs, w1cat, w2cat, b1cat, b2stack, sched):
    row_blk = sched[:W_MAX, 0]
    f_blk = sched[:W_MAX, 1]
    first = sched[:W_MAX, 2]
    last = sched[:W_MAX, 3]
    eid = sched[:W_MAX, 4]
    grid_spec = pltpu.PrefetchScalarGridSpec(
        num_scalar_prefetch=5,
        grid=(W_MAX,),
        in_specs=[
            pl.BlockSpec((BLK, H), lambda g, rb, fb, fi, la, ei: (rb[g], 0)),
            pl.BlockSpec((BF, H), lambda g, rb, fb, fi, la, ei: (fb[g], 0)),
            pl.BlockSpec((BF, H), lambda g, rb, fb, fi, la, ei: (fb[g], 0)),
            pl.BlockSpec((1, 1, BF), lambda g, rb, fb, fi, la, ei: (fb[g], 0, 0)),
            pl.BlockSpec((1, 1, H), lambda g, rb, fb, fi, la, ei: (ei[g], 0, 0)),
        ],
        out_specs=pl.BlockSpec(
            (BLK, H), lambda g, rb, fb, fi, la, ei: (rb[g], 0)),
    )
    return pl.pallas_call(
        _k3_body,
        grid_spec=grid_spec,
        out_shape=jax.ShapeDtypeStruct((R_MAX, H), jnp.float32),
    )(row_blk, f_blk, first, last, eid, xs, w1cat, w2cat, b1cat, b2stack)


K4_CH = 16
K4_NCH = TPW // K4_CH


def _k4_body(y_hbm, pos0_hbm, pos1_hbm, w0_hbm, w1_hbm, out_hbm,
             buf0, buf1, idx, wbuf, sem0, sem1):
    wid = lax.axis_index("s") * NC + lax.axis_index("c")
    base = wid * TPW
    for c in range(K4_NCH):
        pltpu.sync_copy(pos0_hbm.at[pl.ds(base + c * K4_CH, K4_CH)],
                        idx.at[2 * c])
        pltpu.sync_copy(pos1_hbm.at[pl.ds(base + c * K4_CH, K4_CH)],
                        idx.at[2 * c + 1])
    pltpu.sync_copy(w0_hbm.at[pl.ds(base, TPW), :], wbuf.at[0])
    pltpu.sync_copy(w1_hbm.at[pl.ds(base, TPW), :], wbuf.at[1])
    for c in range(K4_NCH):
        cp0 = pltpu.async_copy(y_hbm.at[idx.at[2 * c]], buf0, sem0)
        cp1 = pltpu.async_copy(y_hbm.at[idx.at[2 * c + 1]], buf1, sem1)
        cp0.wait()
        cp1.wait()

        def body(t, _, c=c):
            def inner(v, _):
                s = pl.ds(v * 16, 16)
                buf0[t, s] = (buf0[t, s] * wbuf[0, c * K4_CH + t, :]
                              + buf1[t, s] * wbuf[1, c * K4_CH + t, :])
                return 0

            return lax.fori_loop(0, H // 16, inner, 0, unroll=8)

        lax.fori_loop(0, K4_CH, body, 0)
        pltpu.sync_copy(buf0, out_hbm.at[pl.ds(base + c * K4_CH, K4_CH), :])


def _run_k4(y, pos0, pos1, w0r, w1r):
    mesh = plsc.VectorSubcoreMesh(core_axis_name="c", subcore_axis_name="s")
    return pl.kernel(
        _k4_body,
        mesh=mesh,
        out_type=jax.ShapeDtypeStruct((T, H), jnp.float32),
        scratch_types=[
            pltpu.VMEM((K4_CH, H), jnp.float32),
            pltpu.VMEM((K4_CH, H), jnp.float32),
            pltpu.VMEM((2 * K4_NCH, K4_CH), jnp.int32),
            pltpu.VMEM((2, TPW, 16), jnp.float32),
            pltpu.SemaphoreType.DMA,
            pltpu.SemaphoreType.DMA,
        ],
    )(y, pos0, pos1, w0r, w1r)


def kernel(x, gate_w, gate_b, fc1_w, fc1_b, fc2_w, fc2_b):
    b, s, h = x.shape
    x2d = x.reshape(T, H)

    misc, sched = _run_k1(x2d, gate_w, gate_b)
    pos0 = misc[:, 0].astype(jnp.int32)
    pos1 = misc[:, 1].astype(jnp.int32)
    w0r = jnp.broadcast_to(misc[:, 2:3], (T, 16))
    w1r = jnp.broadcast_to(misc[:, 3:4], (T, 16))

    xs = _run_k2(x2d, pos0, pos1)

    w1cat = jnp.concatenate(fc1_w, axis=0)
    w2cat = jnp.concatenate([w.T for w in fc2_w], axis=0)
    b1cat = jnp.concatenate(fc1_b, axis=0).reshape(NFB, 1, BF)
    b2stack = jnp.stack(fc2_b, axis=0).reshape(E, 1, H)

    y = _run_k3(xs, w1cat, w2cat, b1cat, b2stack, sched)
    out = _run_k4(y, pos0, pos1, w0r, w1r)
    return out.reshape(b, s, h)


# speedup vs baseline: 1.2915x; 1.2856x over previous
import functools

import jax
import jax.numpy as jnp
from jax import lax
from jax.experimental import pallas as pl
from jax.experimental.pallas import tpu as pltpu
from jax.experimental.pallas import tpu_sc as plsc

H = 1024
FF = (512, 768, 1024, 1536, 1536, 1024, 768, 512)
E = 8
T = 2048
TOPK = 2

BLK = 512
BF = 256
KF = tuple(f // BF for f in FF)
FSTARTB = (0, 2, 5, 9, 15, 21, 25, 28)
NFB = sum(KF)
NB_MAX = 15
SCRATCH_BLK = NB_MAX
R_MAX = (NB_MAX + 1) * BLK
W_MAX = 80
LANES = 128

NC = 2
NS = 16
NW = NC * NS
TPW = T // NW


def _k1_body(x_ref, gw_ref, gb_ref, const_ref, misc_ref, w0r_ref, w1r_ref,
             sched_ref):
    f32 = jnp.float32
    x = x_ref[...]
    gw = gw_ref[...]
    logits = lax.dot_general(x.astype(jnp.bfloat16), gw.astype(jnp.bfloat16),
                             (((1,), (1,)), ((), ())),
                             preferred_element_type=f32)
    logits = logits + gb_ref[...]
    lane = lax.broadcasted_iota(jnp.int32, (T, LANES), 1)

    m1 = jnp.max(logits, axis=1, keepdims=True)
    a1 = jnp.min(jnp.where(logits == m1, lane, LANES), axis=1, keepdims=True)
    l2 = jnp.where(lane == a1, -jnp.inf, logits)
    m2 = jnp.max(l2, axis=1, keepdims=True)
    a2 = jnp.min(jnp.where(l2 == m2, lane, LANES), axis=1, keepdims=True)
    e2 = jnp.exp(m2 - m1)
    w1 = 1.0 / (1.0 + e2)
    w2 = e2 / (1.0 + e2)

    m_ind = jnp.logical_or(lane == a1, lane == a2).astype(f32)
    r_iota = lax.broadcasted_iota(jnp.int32, (LANES, LANES), 0)
    c_iota = lax.broadcasted_iota(jnp.int32, (LANES, LANES), 1)
    tril_strict = (c_iota < r_iota).astype(f32)
    triu_strict = (r_iota < c_iota).astype(f32)
    acc = jnp.zeros((1, LANES), f32)
    ranks = []
    for i in range(T // LANES):
        mi = m_ind[i * LANES:(i + 1) * LANES, :]
        ranks.append(lax.dot_general(
            tril_strict, mi, (((1,), (0,)), ((), ())),
            preferred_element_type=f32,
            precision=lax.Precision.HIGHEST) + acc)
        acc = acc + jnp.sum(mi, axis=0, keepdims=True)
    rank = jnp.concatenate(ranks, axis=0)
    counts = acc

    nb = jnp.floor((counts + (BLK - 1)) * (1.0 / BLK))
    sb = lax.dot_general(nb, triu_strict, (((1,), (0,)), ((), ())),
                         preferred_element_type=f32,
                         precision=lax.Precision.HIGHEST)
    kfv = const_ref[0:1, :]
    work = nb * kfv
    ws = lax.dot_general(work, triu_strict, (((1,), (0,)), ((), ())),
                         preferred_element_type=f32,
                         precision=lax.Precision.HIGHEST)
    w_total = jnp.sum(work, axis=1, keepdims=True)

    dest = sb * float(BLK) + rank
    p1 = jnp.sum(jnp.where(lane == a1, dest, 0.0), axis=1, keepdims=True)
    p2 = jnp.sum(jnp.where(lane == a2, dest, 0.0), axis=1, keepdims=True)

    misc = jnp.concatenate(
        [p1, p2, w1, w2, a1.astype(f32), a2.astype(f32),
         jnp.zeros((T, 2), f32)], axis=1)
    sel = (lax.broadcasted_iota(jnp.int32, (8, 8), 0)
           == lax.broadcasted_iota(jnp.int32, (8, 8), 1)).astype(f32)
    misc_ref[...] = lax.dot_general(
        sel, misc, (((1,), (1,)), ((), ())),
        preferred_element_type=f32, precision=lax.Precision.HIGHEST)
    w0r_ref[...] = jnp.broadcast_to(w1, (T, 16))
    w1r_ref[...] = jnp.broadcast_to(w2, (T, 16))

    gi = lax.broadcasted_iota(jnp.int32, (LANES, LANES), 0).astype(f32)
    gcol = gi[:, 0:1]
    elane = lax.broadcasted_iota(jnp.int32, (LANES, LANES), 1)
    ws_b = jnp.broadcast_to(ws, (LANES, LANES))
    in_range = jnp.logical_and(gi >= ws_b, elane < E).astype(f32)
    e_g = jnp.sum(in_range, axis=1, keepdims=True) - 1.0
    onehot = (elane.astype(f32) == e_g).astype(f32)
    ws_g = jnp.sum(onehot * ws_b, axis=1, keepdims=True)
    kf_g = jnp.sum(onehot * kfv, axis=1, keepdims=True)
    sb_g = jnp.sum(onehot * jnp.broadcast_to(sb, (LANES, LANES)),
                   axis=1, keepdims=True)
    fsb = const_ref[1:2, :]
    fsb_g = jnp.sum(onehot * fsb, axis=1, keepdims=True)
    local = gcol - ws_g
    r_g = jnp.floor(local / jnp.maximum(kf_g, 1.0))
    j_g = local - r_g * kf_g
    valid = (gcol < w_total).astype(f32)
    row_blk = jnp.where(valid > 0, sb_g + r_g, float(SCRATCH_BLK))
    f_blk = jnp.where(valid > 0, fsb_g + j_g, 0.0)
    first = jnp.where(valid > 0, (j_g == 0.0).astype(f32), 0.0)
    last = jnp.where(valid > 0, (j_g == kf_g - 1.0).astype(f32), 0.0)
    eid = jnp.where(valid > 0, e_g, 0.0)
    sched_ref[...] = jnp.concatenate(
        [row_blk, f_blk, first, last, eid,
         jnp.zeros((LANES, LANES - 5), f32)], axis=1).astype(jnp.int32)


def _run_k1(x2d, gate_w, gate_b):
    gwp = jnp.zeros((LANES, H), jnp.float32).at[:E].set(gate_w)
    gbp = jnp.full((1, LANES), -1e30, jnp.float32).at[0, :E].set(gate_b)
    consts = jnp.zeros((2, LANES), jnp.float32)
    consts = consts.at[0, :E].set(jnp.asarray(KF, jnp.float32))
    consts = consts.at[1, :E].set(jnp.asarray(FSTARTB, jnp.float32))
    return pl.pallas_call(
        _k1_body,
        out_shape=(jax.ShapeDtypeStruct((8, T), jnp.float32),
                   jax.ShapeDtypeStruct((T, 16), jnp.float32),
                   jax.ShapeDtypeStruct((T, 16), jnp.float32),
                   jax.ShapeDtypeStruct((LANES, LANES), jnp.int32)),
    )(x2d, gwp, gbp, consts)


K2_CH = 32
K2_NCH = TPW // K2_CH


def _k2_body(x_hbm, pos0_hbm, pos1_hbm, xs_hbm, xbuf, idx, sem):
    wid = lax.axis_index("s") * NC + lax.axis_index("c")
    base = wid * TPW
    for c in range(K2_NCH):
        pltpu.sync_copy(pos0_hbm.at[pl.ds(base + c * K2_CH, K2_CH)],
                        idx.at[2 * c])
        pltpu.sync_copy(pos1_hbm.at[pl.ds(base + c * K2_CH, K2_CH)],
                        idx.at[2 * c + 1])
    for c in range(K2_NCH):
        pltpu.sync_copy(x_hbm.at[pl.ds(base + c * K2_CH, K2_CH), :], xbuf)
        cp0 = pltpu.async_copy(xbuf, xs_hbm.at[idx.at[2 * c]], sem)
        cp1 = pltpu.async_copy(xbuf, xs_hbm.at[idx.at[2 * c + 1]], sem)
        cp0.wait()
        cp1.wait()


def _run_k2(x2d, pos0, pos1):
    mesh = plsc.VectorSubcoreMesh(core_axis_name="c", subcore_axis_name="s")
    return pl.kernel(
        _k2_body,
        mesh=mesh,
        out_type=jax.ShapeDtypeStruct((R_MAX, H), jnp.float32),
        scratch_types=[
            pltpu.VMEM((K2_CH, H), jnp.float32),
            pltpu.VMEM((2 * K2_NCH, K2_CH), jnp.int32),
            pltpu.SemaphoreType.DMA,
        ],
    )(x2d, pos0, pos1)


def _k3_body(rb_ref, fb_ref, fi_ref, la_ref, ei_ref,
             x_ref, *rest):
    w1_refs = rest[0:E]
    w2_refs = rest[E:2 * E]
    b1_ref, b2_ref, y_ref, pre_scr, h_scr, y_scr = rest[2 * E:]
    g = pl.program_id(0)
    valid = rb_ref[g] != SCRATCH_BLK
    eid = ei_ref[g]
    first = fi_ref[g] == 1
    last = la_ref[g] == 1

    for e in range(E):
        @pl.when(jnp.logical_and(valid, eid == e))
        def _(e=e):
            xb = x_ref[...].astype(jnp.bfloat16)
            w1 = w1_refs[e][...].astype(jnp.bfloat16)
            pre_scr[...] = lax.dot_general(
                xb, w1, (((1,), (1,)), ((), ())),
                preferred_element_type=jnp.float32)

    @pl.when(valid)
    def _():
        pre = pre_scr[...] + b1_ref[0]
        h = 0.5 * pre * (1.0 + lax.erf(pre * 0.7071067811865475))
        h_scr[...] = h.astype(jnp.bfloat16)

    for e in range(E):
        @pl.when(jnp.logical_and(valid, eid == e))
        def _(e=e):
            w2 = w2_refs[e][...].astype(jnp.bfloat16)
            y_scr[...] = lax.dot_general(
                h_scr[...], w2, (((1,), (1,)), ((), ())),
                preferred_element_type=jnp.float32)

    @pl.when(jnp.logical_and(valid, first))
    def _():
        y_ref[...] = y_scr[...]

    @pl.when(jnp.logical_and(valid, jnp.logical_not(first)))
    def _():
        y_ref[...] = y_ref[...] + y_scr[...]

    @pl.when(jnp.logical_and(valid, last))
    def _():
        y_ref[...] = y_ref[...] + b2_ref[0]


def _run_k3(xs, fc1_w, fc2_w, b1cat, b2stack, sched):
    row_blk = sched[:W_MAX, 0]
    f_blk = sched[:W_MAX, 1]
    first = sched[:W_MAX, 2]
    last = sched[:W_MAX, 3]
    eid = sched[:W_MAX, 4]

    def w1_map(e):
        lo, hi = FSTARTB[e], KF[e] - 1
        return lambda g, rb, fb, fi, la, ei: (
            jnp.clip(fb[g] - lo, 0, hi), 0)

    def w2_map(e):
        lo, hi = FSTARTB[e], KF[e] - 1
        return lambda g, rb, fb, fi, la, ei: (
            0, jnp.clip(fb[g] - lo, 0, hi))

    in_specs = (
        [pl.BlockSpec((BLK, H), lambda g, rb, fb, fi, la, ei: (rb[g], 0))]
        + [pl.BlockSpec((BF, H), w1_map(e)) for e in range(E)]
        + [pl.BlockSpec((H, BF), w2_map(e)) for e in range(E)]
        + [pl.BlockSpec((1, 1, BF),
                        lambda g, rb, fb, fi, la, ei: (fb[g], 0, 0)),
           pl.BlockSpec((1, 1, H),
                        lambda g, rb, fb, fi, la, ei: (ei[g], 0, 0))]
    )
    grid_spec = pltpu.PrefetchScalarGridSpec(
        num_scalar_prefetch=5,
        grid=(W_MAX,),
        in_specs=in_specs,
        out_specs=pl.BlockSpec(
            (BLK, H), lambda g, rb, fb, fi, la, ei: (rb[g], 0)),
        scratch_shapes=[
            pltpu.VMEM((BLK, BF), jnp.float32),
            pltpu.VMEM((BLK, BF), jnp.bfloat16),
            pltpu.VMEM((BLK, H), jnp.float32),
        ],
    )
    return pl.pallas_call(
        _k3_body,
        grid_spec=grid_spec,
        out_shape=jax.ShapeDtypeStruct((R_MAX, H), jnp.float32),
    )(row_blk, f_blk, first, last, eid, xs,
      *fc1_w, *fc2_w, b1cat, b2stack)


K4_CH = 8
K4_NCH = TPW // K4_CH


def _k4_body(y_hbm, pos0_hbm, pos1_hbm, w0_hbm, w1_hbm, out_hbm,
             buf0, buf1, idx, wbuf, sem00, sem01, sem10, sem11):
    wid = lax.axis_index("s") * NC + lax.axis_index("c")
    base = wid * TPW
    sems = ((sem00, sem01), (sem10, sem11))
    for c in range(K4_NCH):
        pltpu.sync_copy(pos0_hbm.at[pl.ds(base + c * K4_CH, K4_CH)],
                        idx.at[2 * c])
        pltpu.sync_copy(pos1_hbm.at[pl.ds(base + c * K4_CH, K4_CH)],
                        idx.at[2 * c + 1])
    pltpu.sync_copy(w0_hbm.at[pl.ds(base, TPW), :], wbuf.at[0])
    pltpu.sync_copy(w1_hbm.at[pl.ds(base, TPW), :], wbuf.at[1])

    def issue(c):
        p = c % 2
        return (pltpu.async_copy(y_hbm.at[idx.at[2 * c]], buf0.at[p],
                                 sems[p][0]),
                pltpu.async_copy(y_hbm.at[idx.at[2 * c + 1]], buf1.at[p],
                                 sems[p][1]))

    cps = {0: issue(0)}
    for c in range(K4_NCH):
        if c + 1 < K4_NCH:
            cps[c + 1] = issue(c + 1)
        cps[c][0].wait()
        cps[c][1].wait()
        p = c % 2
        for t in range(K4_CH):
            wv0 = wbuf[0, c * K4_CH + t, :]
            wv1 = wbuf[1, c * K4_CH + t, :]

            def inner(v, _, p=p, t=t, wv0=wv0, wv1=wv1):
                s = pl.ds(v * 16, 16)
                buf0[p, t, s] = buf0[p, t, s] * wv0 + buf1[p, t, s] * wv1
                return 0

            lax.fori_loop(0, H // 16, inner, 0, unroll=8)
        pltpu.sync_copy(buf0.at[p],
                        out_hbm.at[pl.ds(base + c * K4_CH, K4_CH), :])


def _run_k4(y, pos0, pos1, w0r, w1r):
    mesh = plsc.VectorSubcoreMesh(core_axis_name="c", subcore_axis_name="s")
    return pl.kernel(
        _k4_body,
        mesh=mesh,
        out_type=jax.ShapeDtypeStruct((T, H), jnp.float32),
        scratch_types=[
            pltpu.VMEM((2, K4_CH, H), jnp.float32),
            pltpu.VMEM((2, K4_CH, H), jnp.float32),
            pltpu.VMEM((2 * K4_NCH, K4_CH), jnp.int32),
            pltpu.VMEM((2, TPW, 16), jnp.float32),
            pltpu.SemaphoreType.DMA,
            pltpu.SemaphoreType.DMA,
            pltpu.SemaphoreType.DMA,
            pltpu.SemaphoreType.DMA,
        ],
    )(y, pos0, pos1, w0r, w1r)


def kernel(x, gate_w, gate_b, fc1_w, fc1_b, fc2_w, fc2_b):
    b, s, h = x.shape
    x2d = x.reshape(T, H)

    misc, w0r, w1r, sched = _run_k1(x2d, gate_w, gate_b)
    pos0 = misc[0].astype(jnp.int32)
    pos1 = misc[1].astype(jnp.int32)

    xs = _run_k2(x2d, pos0, pos1)

    b1cat = jnp.concatenate(fc1_b, axis=0).reshape(NFB, 1, BF)
    b2stack = jnp.stack(fc2_b, axis=0).reshape(E, 1, H)

    y = _run_k3(xs, fc1_w, fc2_w, b1cat, b2stack, sched)
    out = _run_k4(y, pos0, pos1, w0r, w1r)
    return out.reshape(b, s, h)


# speedup vs baseline: 1.5395x; 1.1920x over previous
import functools

import jax
import jax.numpy as jnp
from jax import lax
from jax.experimental import pallas as pl
from jax.experimental.pallas import tpu as pltpu
from jax.experimental.pallas import tpu_sc as plsc

H = 1024
FF = (512, 768, 1024, 1536, 1536, 1024, 768, 512)
E = 8
T = 2048
TOPK = 2

BLK = 512
BF = 256
KF = tuple(f // BF for f in FF)
FSTARTB = (0, 2, 5, 9, 15, 21, 25, 28)
NFB = sum(KF)
NB_MAX = 15
SCRATCH_BLK = NB_MAX
R_MAX = (NB_MAX + 1) * BLK
W_MAX = 80
LANES = 128

NC = 2
NS = 16
NW = NC * NS
TPW = T // NW


def _k1_body(x_ref, gw_ref, gb_ref, const_ref, misc_ref, w0r_ref, w1r_ref,
             sched_ref):
    f32 = jnp.float32
    x = x_ref[...]
    gw = gw_ref[...]
    logits = lax.dot_general(x.astype(jnp.bfloat16), gw.astype(jnp.bfloat16),
                             (((1,), (1,)), ((), ())),
                             preferred_element_type=f32)
    logits = logits + gb_ref[...]
    lane = lax.broadcasted_iota(jnp.int32, (T, LANES), 1)

    m1 = jnp.max(logits, axis=1, keepdims=True)
    a1 = jnp.min(jnp.where(logits == m1, lane, LANES), axis=1, keepdims=True)
    l2 = jnp.where(lane == a1, -jnp.inf, logits)
    m2 = jnp.max(l2, axis=1, keepdims=True)
    a2 = jnp.min(jnp.where(l2 == m2, lane, LANES), axis=1, keepdims=True)
    e2 = jnp.exp(m2 - m1)
    w1 = 1.0 / (1.0 + e2)
    w2 = e2 / (1.0 + e2)

    m_ind = jnp.logical_or(lane == a1, lane == a2).astype(f32)
    r_iota = lax.broadcasted_iota(jnp.int32, (LANES, LANES), 0)
    c_iota = lax.broadcasted_iota(jnp.int32, (LANES, LANES), 1)
    tril_strict = (c_iota < r_iota).astype(f32)
    triu_strict = (r_iota < c_iota).astype(f32)
    acc = jnp.zeros((1, LANES), f32)
    ranks = []
    for i in range(T // LANES):
        mi = m_ind[i * LANES:(i + 1) * LANES, :]
        ranks.append(lax.dot_general(
            tril_strict, mi, (((1,), (0,)), ((), ())),
            preferred_element_type=f32,
            precision=lax.Precision.HIGHEST) + acc)
        acc = acc + jnp.sum(mi, axis=0, keepdims=True)
    rank = jnp.concatenate(ranks, axis=0)
    counts = acc

    nb = jnp.floor((counts + (BLK - 1)) * (1.0 / BLK))
    sb = lax.dot_general(nb, triu_strict, (((1,), (0,)), ((), ())),
                         preferred_element_type=f32,
                         precision=lax.Precision.HIGHEST)
    kfv = const_ref[0:1, :]
    work = nb * kfv
    ws = lax.dot_general(work, triu_strict, (((1,), (0,)), ((), ())),
                         preferred_element_type=f32,
                         precision=lax.Precision.HIGHEST)
    w_total = jnp.sum(work, axis=1, keepdims=True)

    dest = sb * float(BLK) + rank
    p1 = jnp.sum(jnp.where(lane == a1, dest, 0.0), axis=1, keepdims=True)
    p2 = jnp.sum(jnp.where(lane == a2, dest, 0.0), axis=1, keepdims=True)

    misc = jnp.concatenate(
        [p1, p2, w1, w2, a1.astype(f32), a2.astype(f32),
         jnp.zeros((T, 2), f32)], axis=1)
    sel = (lax.broadcasted_iota(jnp.int32, (8, 8), 0)
           == lax.broadcasted_iota(jnp.int32, (8, 8), 1)).astype(f32)
    misc_ref[...] = lax.dot_general(
        sel, misc, (((1,), (1,)), ((), ())),
        preferred_element_type=f32, precision=lax.Precision.HIGHEST)
    w0r_ref[...] = jnp.broadcast_to(w1, (T, 16))
    w1r_ref[...] = jnp.broadcast_to(w2, (T, 16))

    gi = lax.broadcasted_iota(jnp.int32, (LANES, LANES), 0).astype(f32)
    gcol = gi[:, 0:1]
    elane = lax.broadcasted_iota(jnp.int32, (LANES, LANES), 1)
    ws_b = jnp.broadcast_to(ws, (LANES, LANES))
    in_range = jnp.logical_and(gi >= ws_b, elane < E).astype(f32)
    e_g = jnp.sum(in_range, axis=1, keepdims=True) - 1.0
    onehot = (elane.astype(f32) == e_g).astype(f32)
    ws_g = jnp.sum(onehot * ws_b, axis=1, keepdims=True)
    kf_g = jnp.sum(onehot * kfv, axis=1, keepdims=True)
    sb_g = jnp.sum(onehot * jnp.broadcast_to(sb, (LANES, LANES)),
                   axis=1, keepdims=True)
    fsb = const_ref[1:2, :]
    fsb_g = jnp.sum(onehot * fsb, axis=1, keepdims=True)
    local = gcol - ws_g
    r_g = jnp.floor(local / jnp.maximum(kf_g, 1.0))
    j_g = local - r_g * kf_g
    valid = (gcol < w_total).astype(f32)
    row_blk = jnp.where(valid > 0, sb_g + r_g, float(SCRATCH_BLK))
    f_blk = jnp.where(valid > 0, fsb_g + j_g, 0.0)
    first = jnp.where(valid > 0, (j_g == 0.0).astype(f32), 0.0)
    last = jnp.where(valid > 0, (j_g == kf_g - 1.0).astype(f32), 0.0)
    eid = jnp.where(valid > 0, e_g, 0.0)
    jloc = jnp.where(valid > 0, j_g, 0.0)
    n_rblocks = jnp.sum(nb, axis=1, keepdims=True)
    sched_ref[...] = jnp.concatenate(
        [row_blk, f_blk, first, last, eid, jloc,
         jnp.broadcast_to(w_total, (LANES, 1)),
         jnp.broadcast_to(n_rblocks, (LANES, 1)),
         jnp.zeros((LANES, LANES - 8), f32)], axis=1).astype(jnp.int32)


def _run_k1(x2d, gate_w, gate_b):
    gwp = jnp.zeros((LANES, H), jnp.float32).at[:E].set(gate_w)
    gbp = jnp.full((1, LANES), -1e30, jnp.float32).at[0, :E].set(gate_b)
    consts = jnp.zeros((2, LANES), jnp.float32)
    consts = consts.at[0, :E].set(jnp.asarray(KF, jnp.float32))
    consts = consts.at[1, :E].set(jnp.asarray(FSTARTB, jnp.float32))
    return pl.pallas_call(
        _k1_body,
        out_shape=(jax.ShapeDtypeStruct((8, T), jnp.float32),
                   jax.ShapeDtypeStruct((T, 16), jnp.float32),
                   jax.ShapeDtypeStruct((T, 16), jnp.float32),
                   jax.ShapeDtypeStruct((LANES, LANES), jnp.int32)),
    )(x2d, gwp, gbp, consts)


K2_CH = 32
K2_NCH = TPW // K2_CH


def _k2_body(x_hbm, pos0_hbm, pos1_hbm, xs_hbm, xbuf, idx, sem):
    wid = lax.axis_index("s") * NC + lax.axis_index("c")
    base = wid * TPW
    for c in range(K2_NCH):
        pltpu.sync_copy(pos0_hbm.at[pl.ds(base + c * K2_CH, K2_CH)],
                        idx.at[2 * c])
        pltpu.sync_copy(pos1_hbm.at[pl.ds(base + c * K2_CH, K2_CH)],
                        idx.at[2 * c + 1])
    for c in range(K2_NCH):
        pltpu.sync_copy(x_hbm.at[pl.ds(base + c * K2_CH, K2_CH), :], xbuf)
        cp0 = pltpu.async_copy(xbuf, xs_hbm.at[idx.at[2 * c]], sem)
        cp1 = pltpu.async_copy(xbuf, xs_hbm.at[idx.at[2 * c + 1]], sem)
        cp0.wait()
        cp1.wait()


def _run_k2(x2d, pos0, pos1):
    mesh = plsc.VectorSubcoreMesh(core_axis_name="c", subcore_axis_name="s")
    return pl.kernel(
        _k2_body,
        mesh=mesh,
        out_type=jax.ShapeDtypeStruct((R_MAX, H), jnp.float32),
        scratch_types=[
            pltpu.VMEM((K2_CH, H), jnp.float32),
            pltpu.VMEM((2 * K2_NCH, K2_CH), jnp.int32),
            pltpu.SemaphoreType.DMA,
        ],
    )(x2d, pos0, pos1)


def _k3_body(rb_ref, fi_ref, la_ref, ei_ref, jl_ref, wt_ref, nrb_ref, fbg_ref,
             x_hbm, *rest):
    w1_hbm = rest[0:E]
    w2_hbm = rest[E:2 * E]
    (b1_ref, b2_ref, y_hbm,
     xbuf, w1buf, w2buf, ybuf, semx, semw1, semw2, semy) = rest[2 * E:]
    w_total = wt_ref[0]
    n_rblocks = nrb_ref[0]

    def issue_w(g, p):
        jl = jl_ref[g]
        ei = ei_ref[g]
        for e in range(E):
            @pl.when(ei == e)
            def _(e=e):
                pltpu.make_async_copy(
                    w1_hbm[e].at[pl.ds(jl * BF, BF), :],
                    w1buf.at[p], semw1).start()
                pltpu.make_async_copy(
                    w2_hbm[e].at[:, pl.ds(jl * BF, BF)],
                    w2buf.at[p], semw2).start()

    pltpu.make_async_copy(x_hbm.at[pl.ds(0, BLK), :], xbuf.at[0],
                          semx).start()
    issue_w(0, 0)

    def step(g, _):
        p = lax.rem(g, 2)
        rb = rb_ref[g]
        first = fi_ref[g] == 1
        last = la_ref[g] == 1
        xs_slot = lax.rem(rb, 2)

        @pl.when(g + 1 < w_total)
        def _():
            issue_w(g + 1, 1 - p)

            @pl.when(rb_ref[g + 1] != rb)
            def _():
                pltpu.make_async_copy(
                    x_hbm.at[pl.ds(rb_ref[g + 1] * BLK, BLK), :],
                    xbuf.at[lax.rem(rb_ref[g + 1], 2)], semx).start()

        @pl.when(jnp.logical_and(first, rb >= 2))
        def _():
            pltpu.make_async_copy(ybuf.at[0], y_hbm.at[pl.ds(0, BLK), :],
                                  semy).wait()

        @pl.when(first)
        def _():
            pltpu.make_async_copy(x_hbm.at[pl.ds(0, BLK), :], xbuf.at[0],
                                  semx).wait()

        pltpu.make_async_copy(w1_hbm[0].at[pl.ds(0, BF), :], w1buf.at[p],
                              semw1).wait()
        pltpu.make_async_copy(w2_hbm[0].at[:, pl.ds(0, BF)], w2buf.at[p],
                              semw2).wait()

        xb = xbuf[pl.ds(xs_slot, 1)][0].astype(jnp.bfloat16)
        w1 = w1buf[pl.ds(p, 1)][0].astype(jnp.bfloat16)
        pre = lax.dot_general(xb, w1, (((1,), (1,)), ((), ())),
                              preferred_element_type=jnp.float32)
        pre = pre + b1_ref[pl.ds(fbg_ref[g], 1), :]
        h = 0.5 * pre * (1.0 + lax.erf(pre * 0.7071067811865475))
        hb = h.astype(jnp.bfloat16)
        w2 = w2buf[pl.ds(p, 1)][0].astype(jnp.bfloat16)
        y = lax.dot_general(hb, w2, (((1,), (1,)), ((), ())),
                            preferred_element_type=jnp.float32)

        @pl.when(first)
        def _():
            ybuf[pl.ds(xs_slot, 1)] = y[None]

        @pl.when(jnp.logical_not(first))
        def _():
            ybuf[pl.ds(xs_slot, 1)] = ybuf[pl.ds(xs_slot, 1)] + y[None]

        @pl.when(last)
        def _():
            ybuf[pl.ds(xs_slot, 1)] = (
                ybuf[pl.ds(xs_slot, 1)]
                + b2_ref[pl.ds(ei_ref[g], 1), :][None])
            pltpu.make_async_copy(ybuf.at[xs_slot],
                                  y_hbm.at[pl.ds(rb * BLK, BLK), :],
                                  semy).start()

        return 0

    lax.fori_loop(0, w_total, step, 0)

    pltpu.make_async_copy(ybuf.at[0], y_hbm.at[pl.ds(0, BLK), :],
                          semy).wait()

    @pl.when(n_rblocks >= 2)
    def _():
        pltpu.make_async_copy(ybuf.at[0], y_hbm.at[pl.ds(0, BLK), :],
                              semy).wait()


def _run_k3(xs, fc1_w, fc2_w, fc1_b, fc2_b, sched):
    row_blk = sched[:W_MAX, 0]
    first = sched[:W_MAX, 2]
    last = sched[:W_MAX, 3]
    eid = sched[:W_MAX, 4]
    jloc = sched[:W_MAX, 5]
    w_total = sched[0:1, 6]
    n_rblocks = sched[0:1, 7]
    f_blk = sched[:W_MAX, 1]

    any_spec = pl.BlockSpec(memory_space=pl.ANY)
    in_specs = (
        [any_spec]
        + [any_spec] * (2 * E)
        + [pl.BlockSpec((NFB, BF), lambda *_: (0, 0)),
           pl.BlockSpec((E, H), lambda *_: (0, 0))]
    )
    grid_spec = pltpu.PrefetchScalarGridSpec(
        num_scalar_prefetch=8,
        grid=(1,),
        in_specs=in_specs,
        out_specs=any_spec,
        scratch_shapes=[
            pltpu.VMEM((2, BLK, H), jnp.float32),
            pltpu.VMEM((2, BF, H), jnp.float32),
            pltpu.VMEM((2, H, BF), jnp.float32),
            pltpu.VMEM((2, BLK, H), jnp.float32),
            pltpu.SemaphoreType.DMA,
            pltpu.SemaphoreType.DMA,
            pltpu.SemaphoreType.DMA,
            pltpu.SemaphoreType.DMA,
        ],
    )
    return pl.pallas_call(
        _k3_body,
        grid_spec=grid_spec,
        out_shape=jax.ShapeDtypeStruct((R_MAX, H), jnp.float32),
    )(row_blk, first, last, eid, jloc, w_total, n_rblocks, f_blk,
      xs, *fc1_w, *fc2_w,
      jnp.concatenate(fc1_b, axis=0).reshape(NFB, BF),
      jnp.stack(fc2_b, axis=0))


K4_CH = 8
K4_NCH = TPW // K4_CH


def _k4_body(y_hbm, pos0_hbm, pos1_hbm, w0_hbm, w1_hbm, out_hbm,
             buf0, buf1, idx, wbuf, sem00, sem01, sem10, sem11):
    wid = lax.axis_index("s") * NC + lax.axis_index("c")
    base = wid * TPW
    sems = ((sem00, sem01), (sem10, sem11))
    for c in range(K4_NCH):
        pltpu.sync_copy(pos0_hbm.at[pl.ds(base + c * K4_CH, K4_CH)],
                        idx.at[2 * c])
        pltpu.sync_copy(pos1_hbm.at[pl.ds(base + c * K4_CH, K4_CH)],
                        idx.at[2 * c + 1])
    pltpu.sync_copy(w0_hbm.at[pl.ds(base, TPW), :], wbuf.at[0])
    pltpu.sync_copy(w1_hbm.at[pl.ds(base, TPW), :], wbuf.at[1])

    def issue(c):
        p = c % 2
        return (pltpu.async_copy(y_hbm.at[idx.at[2 * c]], buf0.at[p],
                                 sems[p][0]),
                pltpu.async_copy(y_hbm.at[idx.at[2 * c + 1]], buf1.at[p],
                                 sems[p][1]))

    cps = {0: issue(0)}
    for c in range(K4_NCH):
        if c + 1 < K4_NCH:
            cps[c + 1] = issue(c + 1)
        cps[c][0].wait()
        cps[c][1].wait()
        p = c % 2
        for t in range(K4_CH):
            wv0 = wbuf[0, c * K4_CH + t, :]
            wv1 = wbuf[1, c * K4_CH + t, :]

            def inner(v, _, p=p, t=t, wv0=wv0, wv1=wv1):
                s = pl.ds(v * 16, 16)
                buf0[p, t, s] = buf0[p, t, s] * wv0 + buf1[p, t, s] * wv1
                return 0

            lax.fori_loop(0, H // 16, inner, 0, unroll=8)
        pltpu.sync_copy(buf0.at[p],
                        out_hbm.at[pl.ds(base + c * K4_CH, K4_CH), :])


def _run_k4(y, pos0, pos1, w0r, w1r):
    mesh = plsc.VectorSubcoreMesh(core_axis_name="c", subcore_axis_name="s")
    return pl.kernel(
        _k4_body,
        mesh=mesh,
        out_type=jax.ShapeDtypeStruct((T, H), jnp.float32),
        scratch_types=[
            pltpu.VMEM((2, K4_CH, H), jnp.float32),
            pltpu.VMEM((2, K4_CH, H), jnp.float32),
            pltpu.VMEM((2 * K4_NCH, K4_CH), jnp.int32),
            pltpu.VMEM((2, TPW, 16), jnp.float32),
            pltpu.SemaphoreType.DMA,
            pltpu.SemaphoreType.DMA,
            pltpu.SemaphoreType.DMA,
            pltpu.SemaphoreType.DMA,
        ],
    )(y, pos0, pos1, w0r, w1r)


def kernel(x, gate_w, gate_b, fc1_w, fc1_b, fc2_w, fc2_b):
    b, s, h = x.shape
    x2d = x.reshape(T, H)

    misc, w0r, w1r, sched = _run_k1(x2d, gate_w, gate_b)
    pos0 = misc[0].astype(jnp.int32)
    pos1 = misc[1].astype(jnp.int32)

    xs = _run_k2(x2d, pos0, pos1)
    y = _run_k3(xs, fc1_w, fc2_w, fc1_b, fc2_b, sched)
    out = _run_k4(y, pos0, pos1, w0r, w1r)
    return out.reshape(b, s, h)


# speedup vs baseline: 1.7168x; 1.1151x over previous
import functools

import jax
import jax.numpy as jnp
from jax import lax
from jax.experimental import pallas as pl
from jax.experimental.pallas import tpu as pltpu
from jax.experimental.pallas import tpu_sc as plsc

H = 1024
FF = (512, 768, 1024, 1536, 1536, 1024, 768, 512)
E = 8
T = 2048
TOPK = 2

BLK = 512
BF = 256
KF = tuple(f // BF for f in FF)
FSTARTB = (0, 2, 5, 9, 15, 21, 25, 28)
NFB = sum(KF)
BF_E = tuple(f // 2 for f in FF)
BF_MAX = max(BF_E)
NB_MAX = 15
SCRATCH_BLK = NB_MAX
R_MAX = (NB_MAX + 1) * BLK
W_MAX = 80
LANES = 128

NC = 2
NS = 16
NW = NC * NS
TPW = T // NW


def _k1_body(x_ref, gw_ref, gb_ref, const_ref, misc_ref, w0r_ref, w1r_ref,
             sched_ref):
    f32 = jnp.float32
    x = x_ref[...]
    gw = gw_ref[...]
    logits = lax.dot_general(x.astype(jnp.bfloat16), gw.astype(jnp.bfloat16),
                             (((1,), (1,)), ((), ())),
                             preferred_element_type=f32)
    logits = logits + gb_ref[...]
    lane = lax.broadcasted_iota(jnp.int32, (T, LANES), 1)

    m1 = jnp.max(logits, axis=1, keepdims=True)
    a1 = jnp.min(jnp.where(logits == m1, lane, LANES), axis=1, keepdims=True)
    l2 = jnp.where(lane == a1, -jnp.inf, logits)
    m2 = jnp.max(l2, axis=1, keepdims=True)
    a2 = jnp.min(jnp.where(l2 == m2, lane, LANES), axis=1, keepdims=True)
    e2 = jnp.exp(m2 - m1)
    w1 = 1.0 / (1.0 + e2)
    w2 = e2 / (1.0 + e2)

    m_ind = jnp.logical_or(lane == a1, lane == a2).astype(f32)
    r_iota = lax.broadcasted_iota(jnp.int32, (LANES, LANES), 0)
    c_iota = lax.broadcasted_iota(jnp.int32, (LANES, LANES), 1)
    tril_strict = (c_iota < r_iota).astype(f32)
    triu_strict = (r_iota < c_iota).astype(f32)
    acc = jnp.zeros((1, LANES), f32)
    ranks = []
    for i in range(T // LANES):
        mi = m_ind[i * LANES:(i + 1) * LANES, :]
        ranks.append(lax.dot_general(
            tril_strict, mi, (((1,), (0,)), ((), ())),
            preferred_element_type=f32,
            precision=lax.Precision.HIGHEST) + acc)
        acc = acc + jnp.sum(mi, axis=0, keepdims=True)
    rank = jnp.concatenate(ranks, axis=0)
    counts = acc

    nb = jnp.floor((counts + (BLK - 1)) * (1.0 / BLK))
    sb = lax.dot_general(nb, triu_strict, (((1,), (0,)), ((), ())),
                         preferred_element_type=f32,
                         precision=lax.Precision.HIGHEST)
    kfv = const_ref[0:1, :]
    work = nb * kfv
    ws = lax.dot_general(work, triu_strict, (((1,), (0,)), ((), ())),
                         preferred_element_type=f32,
                         precision=lax.Precision.HIGHEST)
    w_total = jnp.sum(work, axis=1, keepdims=True)

    dest = sb * float(BLK) + rank
    p1 = jnp.sum(jnp.where(lane == a1, dest, 0.0), axis=1, keepdims=True)
    p2 = jnp.sum(jnp.where(lane == a2, dest, 0.0), axis=1, keepdims=True)

    misc = jnp.concatenate(
        [p1, p2, w1, w2, a1.astype(f32), a2.astype(f32),
         jnp.zeros((T, 2), f32)], axis=1)
    sel = (lax.broadcasted_iota(jnp.int32, (8, 8), 0)
           == lax.broadcasted_iota(jnp.int32, (8, 8), 1)).astype(f32)
    misc_ref[...] = lax.dot_general(
        sel, misc, (((1,), (1,)), ((), ())),
        preferred_element_type=f32, precision=lax.Precision.HIGHEST)
    w0r_ref[...] = jnp.broadcast_to(w1, (T, 16))
    w1r_ref[...] = jnp.broadcast_to(w2, (T, 16))

    gi = lax.broadcasted_iota(jnp.int32, (LANES, LANES), 0).astype(f32)
    gcol = gi[:, 0:1]
    elane = lax.broadcasted_iota(jnp.int32, (LANES, LANES), 1)
    ws_b = jnp.broadcast_to(ws, (LANES, LANES))
    in_range = jnp.logical_and(gi >= ws_b, elane < E).astype(f32)
    e_g = jnp.sum(in_range, axis=1, keepdims=True) - 1.0
    onehot = (elane.astype(f32) == e_g).astype(f32)
    ws_g = jnp.sum(onehot * ws_b, axis=1, keepdims=True)
    kf_g = jnp.sum(onehot * kfv, axis=1, keepdims=True)
    sb_g = jnp.sum(onehot * jnp.broadcast_to(sb, (LANES, LANES)),
                   axis=1, keepdims=True)
    fsb = const_ref[1:2, :]
    fsb_g = jnp.sum(onehot * fsb, axis=1, keepdims=True)
    local = gcol - ws_g
    r_g = jnp.floor(local / jnp.maximum(kf_g, 1.0))
    j_g = local - r_g * kf_g
    valid = (gcol < w_total).astype(f32)
    row_blk = jnp.where(valid > 0, sb_g + r_g, float(SCRATCH_BLK))
    f_blk = jnp.where(valid > 0, fsb_g + j_g, 0.0)
    first = jnp.where(valid > 0, (j_g == 0.0).astype(f32), 0.0)
    last = jnp.where(valid > 0, (j_g == kf_g - 1.0).astype(f32), 0.0)
    eid = jnp.where(valid > 0, e_g, 0.0)
    jloc = jnp.where(valid > 0, j_g, 0.0)
    n_rblocks = jnp.sum(nb, axis=1, keepdims=True)
    sched_ref[...] = jnp.concatenate(
        [row_blk, f_blk, first, last, eid, jloc,
         jnp.broadcast_to(w_total, (LANES, 1)),
         jnp.broadcast_to(n_rblocks, (LANES, 1)),
         jnp.zeros((LANES, LANES - 8), f32)], axis=1).astype(jnp.int32)


def _run_k1(x2d, gate_w, gate_b):
    gwp = jnp.zeros((LANES, H), jnp.float32).at[:E].set(gate_w)
    gbp = jnp.full((1, LANES), -1e30, jnp.float32).at[0, :E].set(gate_b)
    consts = jnp.zeros((2, LANES), jnp.float32)
    consts = consts.at[0, :E].set(2.0)
    consts = consts.at[1, :E].set(jnp.asarray(FSTARTB, jnp.float32))
    return pl.pallas_call(
        _k1_body,
        out_shape=(jax.ShapeDtypeStruct((8, T), jnp.float32),
                   jax.ShapeDtypeStruct((T, 16), jnp.float32),
                   jax.ShapeDtypeStruct((T, 16), jnp.float32),
                   jax.ShapeDtypeStruct((LANES, LANES), jnp.int32)),
    )(x2d, gwp, gbp, consts)


K2_CH = 32
K2_NCH = TPW // K2_CH


def _k2_body(x_hbm, pos0_hbm, pos1_hbm, xs_hbm, xbuf, idx, sem):
    wid = lax.axis_index("s") * NC + lax.axis_index("c")
    base = wid * TPW
    for c in range(K2_NCH):
        pltpu.sync_copy(pos0_hbm.at[pl.ds(base + c * K2_CH, K2_CH)],
                        idx.at[2 * c])
        pltpu.sync_copy(pos1_hbm.at[pl.ds(base + c * K2_CH, K2_CH)],
                        idx.at[2 * c + 1])
    for c in range(K2_NCH):
        pltpu.sync_copy(x_hbm.at[pl.ds(base + c * K2_CH, K2_CH), :], xbuf)
        cp0 = pltpu.async_copy(xbuf, xs_hbm.at[idx.at[2 * c]], sem)
        cp1 = pltpu.async_copy(xbuf, xs_hbm.at[idx.at[2 * c + 1]], sem)
        cp0.wait()
        cp1.wait()


def _run_k2(x2d, pos0, pos1):
    mesh = plsc.VectorSubcoreMesh(core_axis_name="c", subcore_axis_name="s")
    return pl.kernel(
        _k2_body,
        mesh=mesh,
        out_type=jax.ShapeDtypeStruct((R_MAX, H), jnp.float32),
        scratch_types=[
            pltpu.VMEM((K2_CH, H), jnp.float32),
            pltpu.VMEM((2 * K2_NCH, K2_CH), jnp.int32),
            pltpu.SemaphoreType.DMA,
        ],
    )(x2d, pos0, pos1)


def _k3_body(rb_ref, fi_ref, la_ref, ei_ref, jl_ref, wt_ref, nrb_ref,
             x_hbm, *rest):
    w1_hbm = rest[0:E]
    w2_hbm = rest[E:2 * E]
    (b1_ref, b2_ref, y_hbm,
     xbuf, xbb, w1buf, w2buf, ybuf, semx, semw1, semw2, semy) = rest[2 * E:]
    w_total = wt_ref[0]
    n_rblocks = nrb_ref[0]

    def issue_w(g, p):
        jl = jl_ref[g]
        ei = ei_ref[g]
        for e in range(E):
            bfe = BF_E[e]

            @pl.when(ei == e)
            def _(e=e, bfe=bfe):
                pltpu.make_async_copy(
                    w1_hbm[e].at[pl.ds(jl * bfe, bfe), :],
                    w1buf.at[p, pl.ds(0, bfe), :], semw1).start()
                pltpu.make_async_copy(
                    w2_hbm[e].at[:, pl.ds(jl * bfe, bfe)],
                    w2buf.at[p, :, pl.ds(0, bfe)], semw2).start()

    pltpu.make_async_copy(x_hbm.at[pl.ds(0, BLK), :], xbuf.at[0],
                          semx).start()
    issue_w(0, 0)

    def step(g, _):
        p = lax.rem(g, 2)
        rb = rb_ref[g]
        first = fi_ref[g] == 1
        last = la_ref[g] == 1
        ei = ei_ref[g]
        xs_slot = lax.rem(rb, 2)

        @pl.when(g + 1 < w_total)
        def _():
            issue_w(g + 1, 1 - p)

            @pl.when(rb_ref[g + 1] != rb)
            def _():
                pltpu.make_async_copy(
                    x_hbm.at[pl.ds(rb_ref[g + 1] * BLK, BLK), :],
                    xbuf.at[lax.rem(rb_ref[g + 1], 2)], semx).start()

        @pl.when(jnp.logical_and(first, rb >= 2))
        def _():
            pltpu.make_async_copy(ybuf.at[0], y_hbm.at[pl.ds(0, BLK), :],
                                  semy).wait()

        @pl.when(first)
        def _():
            pltpu.make_async_copy(x_hbm.at[pl.ds(0, BLK), :], xbuf.at[0],
                                  semx).wait()
            xbb[pl.ds(xs_slot, 1)] = (
                xbuf[pl.ds(xs_slot, 1)].astype(jnp.bfloat16))

        for e in range(E):
            bfe = BF_E[e]

            @pl.when(ei == e)
            def _(e=e, bfe=bfe):
                pltpu.make_async_copy(
                    w1_hbm[e].at[pl.ds(0, bfe), :],
                    w1buf.at[p, pl.ds(0, bfe), :], semw1).wait()
                pltpu.make_async_copy(
                    w2_hbm[e].at[:, pl.ds(0, bfe)],
                    w2buf.at[p, :, pl.ds(0, bfe)], semw2).wait()

                xb = xbb[pl.ds(xs_slot, 1)][0]
                w1 = w1buf[pl.ds(p, 1), 0:bfe, :][0].astype(jnp.bfloat16)
                pre = lax.dot_general(xb, w1, (((1,), (1,)), ((), ())),
                                      preferred_element_type=jnp.float32)
                pre = pre + b1_ref[e, pl.ds(jl_ref[g], 1), 0:bfe]
                h = 0.5 * pre * (1.0 + lax.erf(pre * 0.7071067811865475))
                hb = h.astype(jnp.bfloat16)
                w2 = w2buf[pl.ds(p, 1), :, 0:bfe][0].astype(jnp.bfloat16)
                y = lax.dot_general(hb, w2, (((1,), (1,)), ((), ())),
                                    preferred_element_type=jnp.float32)

                @pl.when(first)
                def _():
                    ybuf[pl.ds(xs_slot, 1)] = y[None]

                @pl.when(jnp.logical_not(first))
                def _():
                    ybuf[pl.ds(xs_slot, 1)] = ybuf[pl.ds(xs_slot, 1)] + y[None]

        @pl.when(last)
        def _():
            ybuf[pl.ds(xs_slot, 1)] = (
                ybuf[pl.ds(xs_slot, 1)]
                + b2_ref[pl.ds(ei, 1), :][None])
            pltpu.make_async_copy(ybuf.at[xs_slot],
                                  y_hbm.at[pl.ds(rb * BLK, BLK), :],
                                  semy).start()

        return 0

    lax.fori_loop(0, w_total, step, 0)

    pltpu.make_async_copy(ybuf.at[0], y_hbm.at[pl.ds(0, BLK), :],
                          semy).wait()

    @pl.when(n_rblocks >= 2)
    def _():
        pltpu.make_async_copy(ybuf.at[0], y_hbm.at[pl.ds(0, BLK), :],
                              semy).wait()


def _run_k3(xs, fc1_w, fc2_w, fc1_b, fc2_b, sched):
    row_blk = sched[:W_MAX, 0]
    first = sched[:W_MAX, 2]
    last = sched[:W_MAX, 3]
    eid = sched[:W_MAX, 4]
    jloc = sched[:W_MAX, 5]
    w_total = sched[0:1, 6]
    n_rblocks = sched[0:1, 7]

    b1p = jnp.stack([
        jnp.pad(b.reshape(2, -1), ((0, 0), (0, BF_MAX - b.shape[0] // 2)))
        for b in fc1_b])
    b2stack = jnp.stack(fc2_b, axis=0)

    any_spec = pl.BlockSpec(memory_space=pl.ANY)
    in_specs = (
        [any_spec]
        + [any_spec] * (2 * E)
        + [pl.BlockSpec((E, 2, BF_MAX), lambda *_: (0, 0, 0)),
           pl.BlockSpec((E, H), lambda *_: (0, 0))]
    )
    grid_spec = pltpu.PrefetchScalarGridSpec(
        num_scalar_prefetch=7,
        grid=(1,),
        in_specs=in_specs,
        out_specs=any_spec,
        scratch_shapes=[
            pltpu.VMEM((2, BLK, H), jnp.float32),
            pltpu.VMEM((2, BLK, H), jnp.bfloat16),
            pltpu.VMEM((2, BF_MAX, H), jnp.float32),
            pltpu.VMEM((2, H, BF_MAX), jnp.float32),
            pltpu.VMEM((2, BLK, H), jnp.float32),
            pltpu.SemaphoreType.DMA,
            pltpu.SemaphoreType.DMA,
            pltpu.SemaphoreType.DMA,
            pltpu.SemaphoreType.DMA,
        ],
    )
    return pl.pallas_call(
        _k3_body,
        grid_spec=grid_spec,
        out_shape=jax.ShapeDtypeStruct((R_MAX, H), jnp.float32),
    )(row_blk, first, last, eid, jloc, w_total, n_rblocks,
      xs, *fc1_w, *fc2_w, b1p, b2stack)


K4_CH = 8
K4_NCH = TPW // K4_CH


def _k4_body(y_hbm, pos0_hbm, pos1_hbm, w0_hbm, w1_hbm, out_hbm,
             buf0, buf1, idx, wbuf, sem00, sem01, sem10, sem11):
    wid = lax.axis_index("s") * NC + lax.axis_index("c")
    base = wid * TPW
    sems = ((sem00, sem01), (sem10, sem11))
    for c in range(K4_NCH):
        pltpu.sync_copy(pos0_hbm.at[pl.ds(base + c * K4_CH, K4_CH)],
                        idx.at[2 * c])
        pltpu.sync_copy(pos1_hbm.at[pl.ds(base + c * K4_CH, K4_CH)],
                        idx.at[2 * c + 1])
    pltpu.sync_copy(w0_hbm.at[pl.ds(base, TPW), :], wbuf.at[0])
    pltpu.sync_copy(w1_hbm.at[pl.ds(base, TPW), :], wbuf.at[1])

    def issue(c):
        p = c % 2
        return (pltpu.async_copy(y_hbm.at[idx.at[2 * c]], buf0.at[p],
                                 sems[p][0]),
                pltpu.async_copy(y_hbm.at[idx.at[2 * c + 1]], buf1.at[p],
                                 sems[p][1]))

    cps = {0: issue(0)}
    for c in range(K4_NCH):
        if c + 1 < K4_NCH:
            cps[c + 1] = issue(c + 1)
        cps[c][0].wait()
        cps[c][1].wait()
        p = c % 2
        for t in range(K4_CH):
            wv0 = wbuf[0, c * K4_CH + t, :]
            wv1 = wbuf[1, c * K4_CH + t, :]

            def inner(v, _, p=p, t=t, wv0=wv0, wv1=wv1):
                s = pl.ds(v * 16, 16)
                buf0[p, t, s] = buf0[p, t, s] * wv0 + buf1[p, t, s] * wv1
                return 0

            lax.fori_loop(0, H // 16, inner, 0, unroll=8)
        pltpu.sync_copy(buf0.at[p],
                        out_hbm.at[pl.ds(base + c * K4_CH, K4_CH), :])


def _run_k4(y, pos0, pos1, w0r, w1r):
    mesh = plsc.VectorSubcoreMesh(core_axis_name="c", subcore_axis_name="s")
    return pl.kernel(
        _k4_body,
        mesh=mesh,
        out_type=jax.ShapeDtypeStruct((T, H), jnp.float32),
        scratch_types=[
            pltpu.VMEM((2, K4_CH, H), jnp.float32),
            pltpu.VMEM((2, K4_CH, H), jnp.float32),
            pltpu.VMEM((2 * K4_NCH, K4_CH), jnp.int32),
            pltpu.VMEM((2, TPW, 16), jnp.float32),
            pltpu.SemaphoreType.DMA,
            pltpu.SemaphoreType.DMA,
            pltpu.SemaphoreType.DMA,
            pltpu.SemaphoreType.DMA,
        ],
    )(y, pos0, pos1, w0r, w1r)


def kernel(x, gate_w, gate_b, fc1_w, fc1_b, fc2_w, fc2_b):
    b, s, h = x.shape
    x2d = x.reshape(T, H)

    misc, w0r, w1r, sched = _run_k1(x2d, gate_w, gate_b)
    pos0 = misc[0].astype(jnp.int32)
    pos1 = misc[1].astype(jnp.int32)

    xs = _run_k2(x2d, pos0, pos1)
    y = _run_k3(xs, fc1_w, fc2_w, fc1_b, fc2_b, sched)
    out = _run_k4(y, pos0, pos1, w0r, w1r)
    return out.reshape(b, s, h)


# speedup vs baseline: 1.7252x; 1.0049x over previous
import functools

import jax
import jax.numpy as jnp
from jax import lax
from jax.experimental import pallas as pl
from jax.experimental.pallas import tpu as pltpu
from jax.experimental.pallas import tpu_sc as plsc

H = 1024
FF = (512, 768, 1024, 1536, 1536, 1024, 768, 512)
E = 8
T = 2048
TOPK = 2

BLK = 512
BF = 256
KF = tuple(f // BF for f in FF)
FSTARTB = (0, 2, 5, 9, 15, 21, 25, 28)
NFB = sum(KF)
BF_E = tuple(f // 2 for f in FF)
BF_MAX = max(BF_E)
NB_MAX = 15
SCRATCH_BLK = NB_MAX
R_MAX = (NB_MAX + 1) * BLK
W_MAX = 80
LANES = 128

NC = 2
NS = 16
NW = NC * NS
TPW = T // NW


def _k1_body(x_ref, gw_ref, gb_ref, const_ref, misc_ref, w0r_ref, w1r_ref,
             sched_ref):
    f32 = jnp.float32
    x = x_ref[...]
    gw = gw_ref[...]
    logits = lax.dot_general(x.astype(jnp.bfloat16), gw.astype(jnp.bfloat16),
                             (((1,), (1,)), ((), ())),
                             preferred_element_type=f32)
    logits = logits + gb_ref[...]
    lane = lax.broadcasted_iota(jnp.int32, (T, LANES), 1)

    m1 = jnp.max(logits, axis=1, keepdims=True)
    a1 = jnp.min(jnp.where(logits == m1, lane, LANES), axis=1, keepdims=True)
    l2 = jnp.where(lane == a1, -jnp.inf, logits)
    m2 = jnp.max(l2, axis=1, keepdims=True)
    a2 = jnp.min(jnp.where(l2 == m2, lane, LANES), axis=1, keepdims=True)
    e2 = jnp.exp(m2 - m1)
    w1 = 1.0 / (1.0 + e2)
    w2 = e2 / (1.0 + e2)

    m_ind = jnp.logical_or(lane == a1, lane == a2).astype(f32)
    r_iota = lax.broadcasted_iota(jnp.int32, (LANES, LANES), 0)
    c_iota = lax.broadcasted_iota(jnp.int32, (LANES, LANES), 1)
    tril_strict = (c_iota < r_iota).astype(f32)
    triu_strict = (r_iota < c_iota).astype(f32)
    acc = jnp.zeros((1, LANES), f32)
    ranks = []
    for i in range(T // LANES):
        mi = m_ind[i * LANES:(i + 1) * LANES, :]
        ranks.append(lax.dot_general(
            tril_strict, mi, (((1,), (0,)), ((), ())),
            preferred_element_type=f32,
            precision=lax.Precision.HIGHEST) + acc)
        acc = acc + jnp.sum(mi, axis=0, keepdims=True)
    rank = jnp.concatenate(ranks, axis=0)
    counts = acc

    nb = jnp.floor((counts + (BLK - 1)) * (1.0 / BLK))
    sb = lax.dot_general(nb, triu_strict, (((1,), (0,)), ((), ())),
                         preferred_element_type=f32,
                         precision=lax.Precision.HIGHEST)
    kfv = const_ref[0:1, :]
    work = nb * kfv
    ws = lax.dot_general(work, triu_strict, (((1,), (0,)), ((), ())),
                         preferred_element_type=f32,
                         precision=lax.Precision.HIGHEST)
    w_total = jnp.sum(work, axis=1, keepdims=True)

    dest = sb * float(BLK) + rank
    p1 = jnp.sum(jnp.where(lane == a1, dest, 0.0), axis=1, keepdims=True)
    p2 = jnp.sum(jnp.where(lane == a2, dest, 0.0), axis=1, keepdims=True)

    misc = jnp.concatenate(
        [p1, p2, w1, w2, a1.astype(f32), a2.astype(f32),
         jnp.zeros((T, 2), f32)], axis=1)
    sel = (lax.broadcasted_iota(jnp.int32, (8, 8), 0)
           == lax.broadcasted_iota(jnp.int32, (8, 8), 1)).astype(f32)
    misc_ref[...] = lax.dot_general(
        sel, misc, (((1,), (1,)), ((), ())),
        preferred_element_type=f32, precision=lax.Precision.HIGHEST)
    w0r_ref[...] = jnp.broadcast_to(w1, (T, 16))
    w1r_ref[...] = jnp.broadcast_to(w2, (T, 16))

    gi = lax.broadcasted_iota(jnp.int32, (LANES, LANES), 0).astype(f32)
    gcol = gi[:, 0:1]
    elane = lax.broadcasted_iota(jnp.int32, (LANES, LANES), 1)
    ws_b = jnp.broadcast_to(ws, (LANES, LANES))
    in_range = jnp.logical_and(gi >= ws_b, elane < E).astype(f32)
    e_g = jnp.sum(in_range, axis=1, keepdims=True) - 1.0
    onehot = (elane.astype(f32) == e_g).astype(f32)
    ws_g = jnp.sum(onehot * ws_b, axis=1, keepdims=True)
    kf_g = jnp.sum(onehot * kfv, axis=1, keepdims=True)
    sb_g = jnp.sum(onehot * jnp.broadcast_to(sb, (LANES, LANES)),
                   axis=1, keepdims=True)
    fsb = const_ref[1:2, :]
    fsb_g = jnp.sum(onehot * fsb, axis=1, keepdims=True)
    local = gcol - ws_g
    r_g = jnp.floor(local / jnp.maximum(kf_g, 1.0))
    j_g = local - r_g * kf_g
    valid = (gcol < w_total).astype(f32)
    row_blk = jnp.where(valid > 0, sb_g + r_g, float(SCRATCH_BLK))
    f_blk = jnp.where(valid > 0, fsb_g + j_g, 0.0)
    first = jnp.where(valid > 0, (j_g == 0.0).astype(f32), 0.0)
    last = jnp.where(valid > 0, (j_g == kf_g - 1.0).astype(f32), 0.0)
    eid = jnp.where(valid > 0, e_g, 0.0)
    jloc = jnp.where(valid > 0, j_g, 0.0)
    n_rblocks = jnp.sum(nb, axis=1, keepdims=True)
    wskip = jnp.where(valid > 0, (r_g >= 1.0).astype(f32), 0.0)
    sched_ref[...] = jnp.concatenate(
        [row_blk, f_blk, first, last, eid, jloc,
         jnp.broadcast_to(w_total, (LANES, 1)),
         jnp.broadcast_to(n_rblocks, (LANES, 1)), wskip,
         jnp.zeros((LANES, LANES - 9), f32)], axis=1).astype(jnp.int32)


def _run_k1(x2d, gate_w, gate_b):
    gwp = jnp.zeros((LANES, H), jnp.float32).at[:E].set(gate_w)
    gbp = jnp.full((1, LANES), -1e30, jnp.float32).at[0, :E].set(gate_b)
    consts = jnp.zeros((2, LANES), jnp.float32)
    consts = consts.at[0, :E].set(2.0)
    consts = consts.at[1, :E].set(jnp.asarray(FSTARTB, jnp.float32))
    return pl.pallas_call(
        _k1_body,
        out_shape=(jax.ShapeDtypeStruct((8, T), jnp.float32),
                   jax.ShapeDtypeStruct((T, 16), jnp.float32),
                   jax.ShapeDtypeStruct((T, 16), jnp.float32),
                   jax.ShapeDtypeStruct((LANES, LANES), jnp.int32)),
    )(x2d, gwp, gbp, consts)


K2_CH = 32
K2_NCH = TPW // K2_CH


def _k2_body(x_hbm, pos0_hbm, pos1_hbm, xs_hbm, xbuf, idx, sem):
    wid = lax.axis_index("s") * NC + lax.axis_index("c")
    base = wid * TPW
    for c in range(K2_NCH):
        pltpu.sync_copy(pos0_hbm.at[pl.ds(base + c * K2_CH, K2_CH)],
                        idx.at[2 * c])
        pltpu.sync_copy(pos1_hbm.at[pl.ds(base + c * K2_CH, K2_CH)],
                        idx.at[2 * c + 1])
    for c in range(K2_NCH):
        pltpu.sync_copy(x_hbm.at[pl.ds(base + c * K2_CH, K2_CH), :], xbuf)
        cp0 = pltpu.async_copy(xbuf, xs_hbm.at[idx.at[2 * c]], sem)
        cp1 = pltpu.async_copy(xbuf, xs_hbm.at[idx.at[2 * c + 1]], sem)
        cp0.wait()
        cp1.wait()


def _run_k2(x2d, pos0, pos1):
    mesh = plsc.VectorSubcoreMesh(core_axis_name="c", subcore_axis_name="s")
    return pl.kernel(
        _k2_body,
        mesh=mesh,
        out_type=jax.ShapeDtypeStruct((R_MAX, H), jnp.float32),
        scratch_types=[
            pltpu.VMEM((K2_CH, H), jnp.float32),
            pltpu.VMEM((2 * K2_NCH, K2_CH), jnp.int32),
            pltpu.SemaphoreType.DMA,
        ],
    )(x2d, pos0, pos1)


def _k3_body(rb_ref, fi_ref, la_ref, ei_ref, jl_ref, wt_ref, nrb_ref, wsk_ref,
             x_hbm, *rest):
    w1_hbm = rest[0:E]
    w2_hbm = rest[E:2 * E]
    (b1_ref, b2_ref, y_hbm,
     xbuf, xbb, w1buf, w2buf, ybuf, semx, semw1, semw2, semy) = rest[2 * E:]
    w_total = wt_ref[0]
    n_rblocks = nrb_ref[0]

    def issue_w(g, p):
        jl = jl_ref[g]
        ei = ei_ref[g]
        for e in range(E):
            bfe = BF_E[e]

            @pl.when(ei == e)
            def _(e=e, bfe=bfe):
                pltpu.make_async_copy(
                    w1_hbm[e].at[pl.ds(jl * bfe, bfe), :],
                    w1buf.at[p, pl.ds(0, bfe), :], semw1).start()
                pltpu.make_async_copy(
                    w2_hbm[e].at[:, pl.ds(jl * bfe, bfe)],
                    w2buf.at[p, :, pl.ds(0, bfe)], semw2).start()

    pltpu.make_async_copy(x_hbm.at[pl.ds(0, BLK), :], xbuf.at[0],
                          semx).start()
    issue_w(0, 0)

    def step(g, _):
        p = lax.rem(g, 2)
        rb = rb_ref[g]
        first = fi_ref[g] == 1
        last = la_ref[g] == 1
        ei = ei_ref[g]
        xs_slot = lax.rem(rb, 2)

        @pl.when(g + 1 < w_total)
        def _():
            @pl.when(wsk_ref[g + 1] == 0)
            def _():
                issue_w(g + 1, 1 - p)

            @pl.when(rb_ref[g + 1] != rb)
            def _():
                pltpu.make_async_copy(
                    x_hbm.at[pl.ds(rb_ref[g + 1] * BLK, BLK), :],
                    xbuf.at[lax.rem(rb_ref[g + 1], 2)], semx).start()

        @pl.when(jnp.logical_and(first, rb >= 2))
        def _():
            pltpu.make_async_copy(ybuf.at[0], y_hbm.at[pl.ds(0, BLK), :],
                                  semy).wait()

        @pl.when(first)
        def _():
            pltpu.make_async_copy(x_hbm.at[pl.ds(0, BLK), :], xbuf.at[0],
                                  semx).wait()
            xbb[pl.ds(xs_slot, 1)] = (
                xbuf[pl.ds(xs_slot, 1)].astype(jnp.bfloat16))

        for e in range(E):
            bfe = BF_E[e]

            @pl.when(jnp.logical_and(ei == e, wsk_ref[g] == 0))
            def _(e=e, bfe=bfe):
                pltpu.make_async_copy(
                    w1_hbm[e].at[pl.ds(0, bfe), :],
                    w1buf.at[p, pl.ds(0, bfe), :], semw1).wait()
                pltpu.make_async_copy(
                    w2_hbm[e].at[:, pl.ds(0, bfe)],
                    w2buf.at[p, :, pl.ds(0, bfe)], semw2).wait()

        for e in range(E):
            bfe = BF_E[e]

            @pl.when(ei == e)
            def _(e=e, bfe=bfe):
                xb = xbb[pl.ds(xs_slot, 1)][0]
                w1 = w1buf[pl.ds(p, 1), 0:bfe, :][0].astype(jnp.bfloat16)
                pre = lax.dot_general(xb, w1, (((1,), (1,)), ((), ())),
                                      preferred_element_type=jnp.float32)
                pre = pre + b1_ref[e, pl.ds(jl_ref[g], 1), 0:bfe]
                h = 0.5 * pre * (1.0 + lax.erf(pre * 0.7071067811865475))
                hb = h.astype(jnp.bfloat16)
                w2 = w2buf[pl.ds(p, 1), :, 0:bfe][0].astype(jnp.bfloat16)
                y = lax.dot_general(hb, w2, (((1,), (1,)), ((), ())),
                                    preferred_element_type=jnp.float32)

                @pl.when(first)
                def _():
                    ybuf[pl.ds(xs_slot, 1)] = y[None]

                @pl.when(jnp.logical_not(first))
                def _():
                    ybuf[pl.ds(xs_slot, 1)] = ybuf[pl.ds(xs_slot, 1)] + y[None]

        @pl.when(last)
        def _():
            ybuf[pl.ds(xs_slot, 1)] = (
                ybuf[pl.ds(xs_slot, 1)]
                + b2_ref[pl.ds(ei, 1), :][None])
            pltpu.make_async_copy(ybuf.at[xs_slot],
                                  y_hbm.at[pl.ds(rb * BLK, BLK), :],
                                  semy).start()

        return 0

    lax.fori_loop(0, w_total, step, 0)

    pltpu.make_async_copy(ybuf.at[0], y_hbm.at[pl.ds(0, BLK), :],
                          semy).wait()

    @pl.when(n_rblocks >= 2)
    def _():
        pltpu.make_async_copy(ybuf.at[0], y_hbm.at[pl.ds(0, BLK), :],
                              semy).wait()


def _run_k3(xs, fc1_w, fc2_w, fc1_b, fc2_b, sched):
    row_blk = sched[:W_MAX, 0]
    first = sched[:W_MAX, 2]
    last = sched[:W_MAX, 3]
    eid = sched[:W_MAX, 4]
    jloc = sched[:W_MAX, 5]
    w_total = sched[0:1, 6]
    n_rblocks = sched[0:1, 7]
    wskip = sched[:W_MAX, 8]

    b1p = jnp.stack([
        jnp.pad(b.reshape(2, -1), ((0, 0), (0, BF_MAX - b.shape[0] // 2)))
        for b in fc1_b])
    b2stack = jnp.stack(fc2_b, axis=0)

    any_spec = pl.BlockSpec(memory_space=pl.ANY)
    in_specs = (
        [any_spec]
        + [any_spec] * (2 * E)
        + [pl.BlockSpec((E, 2, BF_MAX), lambda *_: (0, 0, 0)),
           pl.BlockSpec((E, H), lambda *_: (0, 0))]
    )
    grid_spec = pltpu.PrefetchScalarGridSpec(
        num_scalar_prefetch=8,
        grid=(1,),
        in_specs=in_specs,
        out_specs=any_spec,
        scratch_shapes=[
            pltpu.VMEM((2, BLK, H), jnp.float32),
            pltpu.VMEM((2, BLK, H), jnp.bfloat16),
            pltpu.VMEM((2, BF_MAX, H), jnp.float32),
            pltpu.VMEM((2, H, BF_MAX), jnp.float32),
            pltpu.VMEM((2, BLK, H), jnp.float32),
            pltpu.SemaphoreType.DMA,
            pltpu.SemaphoreType.DMA,
            pltpu.SemaphoreType.DMA,
            pltpu.SemaphoreType.DMA,
        ],
    )
    return pl.pallas_call(
        _k3_body,
        grid_spec=grid_spec,
        out_shape=jax.ShapeDtypeStruct((R_MAX, H), jnp.float32),
    )(row_blk, first, last, eid, jloc, w_total, n_rblocks, wskip,
      xs, *fc1_w, *fc2_w, b1p, b2stack)


K4_CH = 8
K4_NCH = TPW // K4_CH


def _k4_body(y_hbm, pos0_hbm, pos1_hbm, w0_hbm, w1_hbm, out_hbm,
             buf0, buf1, idx, wbuf, sem00, sem01, sem10, sem11):
    wid = lax.axis_index("s") * NC + lax.axis_index("c")
    base = wid * TPW
    sems = ((sem00, sem01), (sem10, sem11))
    for c in range(K4_NCH):
        pltpu.sync_copy(pos0_hbm.at[pl.ds(base + c * K4_CH, K4_CH)],
                        idx.at[2 * c])
        pltpu.sync_copy(pos1_hbm.at[pl.ds(base + c * K4_CH, K4_CH)],
                        idx.at[2 * c + 1])
    pltpu.sync_copy(w0_hbm.at[pl.ds(base, TPW), :], wbuf.at[0])
    pltpu.sync_copy(w1_hbm.at[pl.ds(base, TPW), :], wbuf.at[1])

    def issue(c):
        p = c % 2
        return (pltpu.async_copy(y_hbm.at[idx.at[2 * c]], buf0.at[p],
                                 sems[p][0]),
                pltpu.async_copy(y_hbm.at[idx.at[2 * c + 1]], buf1.at[p],
                                 sems[p][1]))

    cps = {0: issue(0)}
    for c in range(K4_NCH):
        if c + 1 < K4_NCH:
            cps[c + 1] = issue(c + 1)
        cps[c][0].wait()
        cps[c][1].wait()
        p = c % 2
        for t in range(K4_CH):
            wv0 = wbuf[0, c * K4_CH + t, :]
            wv1 = wbuf[1, c * K4_CH + t, :]

            def inner(v, _, p=p, t=t, wv0=wv0, wv1=wv1):
                s = pl.ds(v * 16, 16)
                buf0[p, t, s] = buf0[p, t, s] * wv0 + buf1[p, t, s] * wv1
                return 0

            lax.fori_loop(0, H // 16, inner, 0, unroll=8)
        pltpu.sync_copy(buf0.at[p],
                        out_hbm.at[pl.ds(base + c * K4_CH, K4_CH), :])


def _run_k4(y, pos0, pos1, w0r, w1r):
    mesh = plsc.VectorSubcoreMesh(core_axis_name="c", subcore_axis_name="s")
    return pl.kernel(
        _k4_body,
        mesh=mesh,
        out_type=jax.ShapeDtypeStruct((T, H), jnp.float32),
        scratch_types=[
            pltpu.VMEM((2, K4_CH, H), jnp.float32),
            pltpu.VMEM((2, K4_CH, H), jnp.float32),
            pltpu.VMEM((2 * K4_NCH, K4_CH), jnp.int32),
            pltpu.VMEM((2, TPW, 16), jnp.float32),
            pltpu.SemaphoreType.DMA,
            pltpu.SemaphoreType.DMA,
            pltpu.SemaphoreType.DMA,
            pltpu.SemaphoreType.DMA,
        ],
    )(y, pos0, pos1, w0r, w1r)


def kernel(x, gate_w, gate_b, fc1_w, fc1_b, fc2_w, fc2_b):
    b, s, h = x.shape
    x2d = x.reshape(T, H)

    misc, w0r, w1r, sched = _run_k1(x2d, gate_w, gate_b)
    pos0 = misc[0].astype(jnp.int32)
    pos1 = misc[1].astype(jnp.int32)

    xs = _run_k2(x2d, pos0, pos1)
    y = _run_k3(xs, fc1_w, fc2_w, fc1_b, fc2_b, sched)
    out = _run_k4(y, pos0, pos1, w0r, w1r)
    return out.reshape(b, s, h)


# speedup vs baseline: 1.7872x; 1.0359x over previous
import functools

import jax
import jax.numpy as jnp
from jax import lax
from jax.experimental import pallas as pl
from jax.experimental.pallas import tpu as pltpu
from jax.experimental.pallas import tpu_sc as plsc

H = 1024
FF = (512, 768, 1024, 1536, 1536, 1024, 768, 512)
E = 8
T = 2048
TOPK = 2
H2 = H // 2

BLK = 512
BF = 256
KF = tuple(f // BF for f in FF)
FSTARTB = (0, 2, 5, 9, 15, 21, 25, 28)
NFB = sum(KF)
BF_E = tuple(f // 2 for f in FF)
BF_MAX = max(BF_E)
NB_MAX = 15
SCRATCH_BLK = NB_MAX
R_MAX = (NB_MAX + 1) * BLK
W_MAX = 80
LANES = 128

NC = 2
NS = 16
NW = NC * NS
TPW = T // NW


def _k1_body(x_ref, gw_ref, gb_ref, const_ref, misc_ref, w0r_ref, w1r_ref,
             sched_ref):
    f32 = jnp.float32
    x = x_ref[...]
    gw = gw_ref[...]
    logits = lax.dot_general(x.astype(jnp.bfloat16), gw.astype(jnp.bfloat16),
                             (((1,), (1,)), ((), ())),
                             preferred_element_type=f32)
    logits = logits + gb_ref[...]
    lane = lax.broadcasted_iota(jnp.int32, (T, LANES), 1)

    m1 = jnp.max(logits, axis=1, keepdims=True)
    a1 = jnp.min(jnp.where(logits == m1, lane, LANES), axis=1, keepdims=True)
    l2 = jnp.where(lane == a1, -jnp.inf, logits)
    m2 = jnp.max(l2, axis=1, keepdims=True)
    a2 = jnp.min(jnp.where(l2 == m2, lane, LANES), axis=1, keepdims=True)
    e2 = jnp.exp(m2 - m1)
    w1 = 1.0 / (1.0 + e2)
    w2 = e2 / (1.0 + e2)

    m_ind = jnp.logical_or(lane == a1, lane == a2).astype(f32)
    r_iota = lax.broadcasted_iota(jnp.int32, (LANES, LANES), 0)
    c_iota = lax.broadcasted_iota(jnp.int32, (LANES, LANES), 1)
    tril_strict = (c_iota < r_iota).astype(f32)
    triu_strict = (r_iota < c_iota).astype(f32)
    acc = jnp.zeros((1, LANES), f32)
    ranks = []
    for i in range(T // LANES):
        mi = m_ind[i * LANES:(i + 1) * LANES, :]
        ranks.append(lax.dot_general(
            tril_strict, mi, (((1,), (0,)), ((), ())),
            preferred_element_type=f32,
            precision=lax.Precision.HIGHEST) + acc)
        acc = acc + jnp.sum(mi, axis=0, keepdims=True)
    rank = jnp.concatenate(ranks, axis=0)
    counts = acc

    nb = jnp.floor((counts + (BLK - 1)) * (1.0 / BLK))
    sb = lax.dot_general(nb, triu_strict, (((1,), (0,)), ((), ())),
                         preferred_element_type=f32,
                         precision=lax.Precision.HIGHEST)
    kfv = const_ref[0:1, :]
    work = nb * kfv
    ws = lax.dot_general(work, triu_strict, (((1,), (0,)), ((), ())),
                         preferred_element_type=f32,
                         precision=lax.Precision.HIGHEST)
    w_total = jnp.sum(work, axis=1, keepdims=True)

    dest = sb * float(BLK) + rank
    p1 = jnp.sum(jnp.where(lane == a1, dest, 0.0), axis=1, keepdims=True)
    p2 = jnp.sum(jnp.where(lane == a2, dest, 0.0), axis=1, keepdims=True)

    misc = jnp.concatenate(
        [p1, p2, w1, w2, a1.astype(f32), a2.astype(f32),
         jnp.zeros((T, 2), f32)], axis=1)
    sel = (lax.broadcasted_iota(jnp.int32, (8, 8), 0)
           == lax.broadcasted_iota(jnp.int32, (8, 8), 1)).astype(f32)
    misc_ref[...] = lax.dot_general(
        sel, misc, (((1,), (1,)), ((), ())),
        preferred_element_type=f32, precision=lax.Precision.HIGHEST)
    w0r_ref[...] = jnp.broadcast_to(w1, (T, LANES))
    w1r_ref[...] = jnp.broadcast_to(w2, (T, LANES))

    gi = lax.broadcasted_iota(jnp.int32, (LANES, LANES), 0).astype(f32)
    gcol = gi[:, 0:1]
    elane = lax.broadcasted_iota(jnp.int32, (LANES, LANES), 1)
    ws_b = jnp.broadcast_to(ws, (LANES, LANES))
    in_range = jnp.logical_and(gi >= ws_b, elane < E).astype(f32)
    e_g = jnp.sum(in_range, axis=1, keepdims=True) - 1.0
    onehot = (elane.astype(f32) == e_g).astype(f32)
    ws_g = jnp.sum(onehot * ws_b, axis=1, keepdims=True)
    kf_g = jnp.sum(onehot * kfv, axis=1, keepdims=True)
    sb_g = jnp.sum(onehot * jnp.broadcast_to(sb, (LANES, LANES)),
                   axis=1, keepdims=True)
    fsb = const_ref[1:2, :]
    fsb_g = jnp.sum(onehot * fsb, axis=1, keepdims=True)
    local = gcol - ws_g
    r_g = jnp.floor(local / jnp.maximum(kf_g, 1.0))
    j_g = local - r_g * kf_g
    valid = (gcol < w_total).astype(f32)
    row_blk = jnp.where(valid > 0, sb_g + r_g, float(SCRATCH_BLK))
    f_blk = jnp.where(valid > 0, fsb_g + j_g, 0.0)
    first = jnp.where(valid > 0, (j_g == 0.0).astype(f32), 0.0)
    last = jnp.where(valid > 0, (j_g == kf_g - 1.0).astype(f32), 0.0)
    eid = jnp.where(valid > 0, e_g, 0.0)
    jloc = jnp.where(valid > 0, j_g, 0.0)
    n_rblocks = jnp.sum(nb, axis=1, keepdims=True)
    wskip = jnp.where(valid > 0, (r_g >= 1.0).astype(f32), 0.0)
    sched_ref[...] = jnp.concatenate(
        [row_blk, f_blk, first, last, eid, jloc,
         jnp.broadcast_to(w_total, (LANES, 1)),
         jnp.broadcast_to(n_rblocks, (LANES, 1)), wskip,
         jnp.zeros((LANES, LANES - 9), f32)], axis=1).astype(jnp.int32)


def _run_k1(x2d, gate_w, gate_b):
    gwp = jnp.zeros((LANES, H), jnp.float32).at[:E].set(gate_w)
    gbp = jnp.full((1, LANES), -1e30, jnp.float32).at[0, :E].set(gate_b)
    consts = jnp.zeros((2, LANES), jnp.float32)
    consts = consts.at[0, :E].set(2.0)
    consts = consts.at[1, :E].set(jnp.asarray(FSTARTB, jnp.float32))
    return pl.pallas_call(
        _k1_body,
        out_shape=(jax.ShapeDtypeStruct((8, T), jnp.float32),
                   jax.ShapeDtypeStruct((T, LANES), jnp.float32),
                   jax.ShapeDtypeStruct((T, LANES), jnp.float32),
                   jax.ShapeDtypeStruct((LANES, LANES), jnp.int32)),
    )(x2d, gwp, gbp, consts)


K2_CH = 32
K2_NCH = TPW // K2_CH


def _k2_body(x_hbm, pos0_hbm, pos1_hbm, w0_hbm, w1_hbm, xs_hbm, ws_hbm,
             xbuf, wbuf, idx, sem):
    wid = lax.axis_index("s") * NC + lax.axis_index("c")
    base = wid * TPW
    for c in range(K2_NCH):
        pltpu.sync_copy(pos0_hbm.at[pl.ds(base + c * K2_CH, K2_CH)],
                        idx.at[2 * c])
        pltpu.sync_copy(pos1_hbm.at[pl.ds(base + c * K2_CH, K2_CH)],
                        idx.at[2 * c + 1])
    for c in range(K2_NCH):
        pltpu.sync_copy(x_hbm.at[pl.ds(base + c * K2_CH, K2_CH), :], xbuf)
        cp0 = pltpu.async_copy(xbuf, xs_hbm.at[idx.at[2 * c]], sem)
        cp1 = pltpu.async_copy(xbuf, xs_hbm.at[idx.at[2 * c + 1]], sem)
        pltpu.sync_copy(w0_hbm.at[pl.ds(base + c * K2_CH, K2_CH), :], wbuf)
        cp2 = pltpu.async_copy(wbuf, ws_hbm.at[idx.at[2 * c]], sem)
        cp2.wait()
        pltpu.sync_copy(w1_hbm.at[pl.ds(base + c * K2_CH, K2_CH), :], wbuf)
        cp3 = pltpu.async_copy(wbuf, ws_hbm.at[idx.at[2 * c + 1]], sem)
        cp0.wait()
        cp1.wait()
        cp3.wait()


def _run_k2(x2d, pos0, pos1, w0r, w1r):
    mesh = plsc.VectorSubcoreMesh(core_axis_name="c", subcore_axis_name="s")
    return pl.kernel(
        _k2_body,
        mesh=mesh,
        out_type=(jax.ShapeDtypeStruct((R_MAX, H), jnp.float32),
                  jax.ShapeDtypeStruct((R_MAX, LANES), jnp.float32)),
        scratch_types=[
            pltpu.VMEM((K2_CH, H), jnp.float32),
            pltpu.VMEM((K2_CH, LANES), jnp.float32),
            pltpu.VMEM((2 * K2_NCH, K2_CH), jnp.int32),
            pltpu.SemaphoreType.DMA,
        ],
    )(x2d, pos0, pos1, w0r, w1r)


def _k3_body(rb_ref, fi_ref, la_ref, ei_ref, jl_ref, wt_ref, nrb_ref, wsk_ref,
             x_hbm, *rest):
    w1_hbm = rest[0:E]
    w2_hbm = rest[E:2 * E]
    (ws_hbm, b1_ref, b2_ref, y_hbm, xbuf, xbb, w1buf, w2buf, ybuf,
     wrow, semx, semw1, semw2, semy) = rest[2 * E:]
    w_total = wt_ref[0]
    n_rblocks = nrb_ref[0]

    def issue_w(g, p):
        jl = jl_ref[g]
        ei = ei_ref[g]
        for e in range(E):
            bfe = BF_E[e]

            @pl.when(ei == e)
            def _(e=e, bfe=bfe):
                pltpu.make_async_copy(
                    w1_hbm[e].at[pl.ds(jl * bfe, bfe), :],
                    w1buf.at[p, pl.ds(0, bfe), :], semw1).start()
                pltpu.make_async_copy(
                    w2_hbm[e].at[:, pl.ds(jl * bfe, bfe)],
                    w2buf.at[p, :, pl.ds(0, bfe)], semw2).start()

    pltpu.make_async_copy(x_hbm.at[pl.ds(0, BLK), :], xbuf.at[0],
                          semx).start()
    pltpu.make_async_copy(ws_hbm.at[pl.ds(0, BLK), :], wrow.at[0],
                          semx).start()
    issue_w(0, 0)

    def step(g, _):
        p = lax.rem(g, 2)
        rb = rb_ref[g]
        first = fi_ref[g] == 1
        last = la_ref[g] == 1
        ei = ei_ref[g]
        xs_slot = lax.rem(rb, 2)

        @pl.when(g + 1 < w_total)
        def _():
            @pl.when(wsk_ref[g + 1] == 0)
            def _():
                issue_w(g + 1, 1 - p)

            @pl.when(rb_ref[g + 1] != rb)
            def _():
                nslot = lax.rem(rb_ref[g + 1], 2)
                pltpu.make_async_copy(
                    x_hbm.at[pl.ds(rb_ref[g + 1] * BLK, BLK), :],
                    xbuf.at[nslot], semx).start()
                pltpu.make_async_copy(
                    ws_hbm.at[pl.ds(rb_ref[g + 1] * BLK, BLK), :],
                    wrow.at[nslot], semx).start()

        @pl.when(jnp.logical_and(first, rb >= 2))
        def _():
            pltpu.make_async_copy(ybuf.at[0], y_hbm.at[pl.ds(0, BLK), :],
                                  semy).wait()

        @pl.when(first)
        def _():
            pltpu.make_async_copy(x_hbm.at[pl.ds(0, BLK), :], xbuf.at[0],
                                  semx).wait()
            pltpu.make_async_copy(ws_hbm.at[pl.ds(0, BLK), :], wrow.at[0],
                                  semx).wait()
            xbb[pl.ds(xs_slot, 1)] = (
                xbuf[pl.ds(xs_slot, 1)].astype(jnp.bfloat16))

        for e in range(E):
            bfe = BF_E[e]

            @pl.when(jnp.logical_and(ei == e, wsk_ref[g] == 0))
            def _(e=e, bfe=bfe):
                pltpu.make_async_copy(
                    w1_hbm[e].at[pl.ds(0, bfe), :],
                    w1buf.at[p, pl.ds(0, bfe), :], semw1).wait()
                pltpu.make_async_copy(
                    w2_hbm[e].at[:, pl.ds(0, bfe)],
                    w2buf.at[p, :, pl.ds(0, bfe)], semw2).wait()

        for e in range(E):
            bfe = BF_E[e]

            @pl.when(ei == e)
            def _(e=e, bfe=bfe):
                xb = xbb[pl.ds(xs_slot, 1)][0]
                w1 = w1buf[pl.ds(p, 1), 0:bfe, :][0].astype(jnp.bfloat16)
                pre = lax.dot_general(xb, w1, (((1,), (1,)), ((), ())),
                                      preferred_element_type=jnp.float32)
                pre = pre + b1_ref[e, pl.ds(jl_ref[g], 1), 0:bfe]
                h = 0.5 * pre * (1.0 + lax.erf(pre * 0.7071067811865475))
                hb = h.astype(jnp.bfloat16)
                w2 = w2buf[pl.ds(p, 1), :, 0:bfe][0].astype(jnp.bfloat16)
                y = lax.dot_general(hb, w2, (((1,), (1,)), ((), ())),
                                    preferred_element_type=jnp.float32)

                @pl.when(first)
                def _():
                    ybuf[pl.ds(xs_slot, 1)] = y[None]

                @pl.when(jnp.logical_not(first))
                def _():
                    ybuf[pl.ds(xs_slot, 1)] = ybuf[pl.ds(xs_slot, 1)] + y[None]

        @pl.when(last)
        def _():
            ybuf[pl.ds(xs_slot, 1)] = (
                (ybuf[pl.ds(xs_slot, 1)] + b2_ref[pl.ds(ei, 1), :][None])
                * wrow[pl.ds(xs_slot, 1), :, 0:1])
            pltpu.make_async_copy(ybuf.at[xs_slot],
                                  y_hbm.at[pl.ds(rb * BLK, BLK), :],
                                  semy).start()

        return 0

    lax.fori_loop(0, w_total, step, 0)

    pltpu.make_async_copy(ybuf.at[0], y_hbm.at[pl.ds(0, BLK), :],
                          semy).wait()

    @pl.when(n_rblocks >= 2)
    def _():
        pltpu.make_async_copy(ybuf.at[0], y_hbm.at[pl.ds(0, BLK), :],
                              semy).wait()


def _run_k3(xs, ws, fc1_w, fc2_w, fc1_b, fc2_b, sched):
    row_blk = sched[:W_MAX, 0]
    first = sched[:W_MAX, 2]
    last = sched[:W_MAX, 3]
    eid = sched[:W_MAX, 4]
    jloc = sched[:W_MAX, 5]
    w_total = sched[0:1, 6]
    n_rblocks = sched[0:1, 7]
    wskip = sched[:W_MAX, 8]

    b1p = jnp.stack([
        jnp.pad(b.reshape(2, -1), ((0, 0), (0, BF_MAX - b.shape[0] // 2)))
        for b in fc1_b])
    b2stack = jnp.stack(fc2_b, axis=0)

    any_spec = pl.BlockSpec(memory_space=pl.ANY)
    in_specs = (
        [any_spec]
        + [any_spec] * (2 * E)
        + [any_spec,
           pl.BlockSpec((E, 2, BF_MAX), lambda *_: (0, 0, 0)),
           pl.BlockSpec((E, H), lambda *_: (0, 0))]
    )
    grid_spec = pltpu.PrefetchScalarGridSpec(
        num_scalar_prefetch=8,
        grid=(1,),
        in_specs=in_specs,
        out_specs=any_spec,
        scratch_shapes=[
            pltpu.VMEM((2, BLK, H), jnp.float32),
            pltpu.VMEM((2, BLK, H), jnp.bfloat16),
            pltpu.VMEM((2, BF_MAX, H), jnp.float32),
            pltpu.VMEM((2, H, BF_MAX), jnp.float32),
            pltpu.VMEM((2, BLK, H), jnp.float32),
            pltpu.VMEM((2, BLK, LANES), jnp.float32),
            pltpu.SemaphoreType.DMA,
            pltpu.SemaphoreType.DMA,
            pltpu.SemaphoreType.DMA,
            pltpu.SemaphoreType.DMA,
        ],
    )
    return pl.pallas_call(
        _k3_body,
        grid_spec=grid_spec,
        out_shape=jax.ShapeDtypeStruct((R_MAX, H), jnp.float32),
    )(row_blk, first, last, eid, jloc, w_total, n_rblocks, wskip,
      xs, *fc1_w, *fc2_w, ws, b1p, b2stack)


K4_CH = 8
K4_NCH = TPW // K4_CH


def _k4_body(y_hbm, pos0_hbm, pos1_hbm, out_hbm,
             buf0, buf1, obuf, idx, sem00, sem01, sem10, sem11):
    wid = lax.axis_index("s") * NC + lax.axis_index("c")
    base = wid * TPW
    sems = ((sem00, sem01), (sem10, sem11))
    for c in range(K4_NCH):
        pltpu.sync_copy(pos0_hbm.at[pl.ds(base + c * K4_CH, K4_CH)],
                        idx.at[2 * c])
        pltpu.sync_copy(pos1_hbm.at[pl.ds(base + c * K4_CH, K4_CH)],
                        idx.at[2 * c + 1])

    def issue(c):
        p = c % 2
        return (pltpu.async_copy(y_hbm.at[idx.at[2 * c]], buf0.at[p],
                                 sems[p][0]),
                pltpu.async_copy(y_hbm.at[idx.at[2 * c + 1]], buf1.at[p],
                                 sems[p][1]))

    cps = {0: issue(0)}
    for c in range(K4_NCH):
        if c + 1 < K4_NCH:
            cps[c + 1] = issue(c + 1)
        cps[c][0].wait()
        cps[c][1].wait()
        p = c % 2
        def tbody(t, _, p=p):
            def inner(v, _):
                s = pl.ds(v * 16, 16)
                obuf[t, s] = buf0[p, t, s] + buf1[p, t, s]
                return 0

            return lax.fori_loop(0, H // 16, inner, 0, unroll=8)

        lax.fori_loop(0, K4_CH, tbody, 0)
        pltpu.sync_copy(obuf,
                        out_hbm.at[pl.ds(base + c * K4_CH, K4_CH), :])


def _run_k4(y, pos0, pos1):
    mesh = plsc.VectorSubcoreMesh(core_axis_name="c", subcore_axis_name="s")
    return pl.kernel(
        _k4_body,
        mesh=mesh,
        out_type=jax.ShapeDtypeStruct((T, H), jnp.float32),
        scratch_types=[
            pltpu.VMEM((2, K4_CH, H), jnp.float32),
            pltpu.VMEM((2, K4_CH, H), jnp.float32),
            pltpu.VMEM((K4_CH, H), jnp.float32),
            pltpu.VMEM((2 * K4_NCH, K4_CH), jnp.int32),
            pltpu.SemaphoreType.DMA,
            pltpu.SemaphoreType.DMA,
            pltpu.SemaphoreType.DMA,
            pltpu.SemaphoreType.DMA,
        ],
    )(y, pos0, pos1)


def kernel(x, gate_w, gate_b, fc1_w, fc1_b, fc2_w, fc2_b):
    b, s, h = x.shape
    x2d = x.reshape(T, H)

    misc, w0r, w1r, sched = _run_k1(x2d, gate_w, gate_b)
    pos0 = misc[0].astype(jnp.int32)
    pos1 = misc[1].astype(jnp.int32)

    xs, ws = _run_k2(x2d, pos0, pos1, w0r, w1r)
    y = _run_k3(xs, ws, fc1_w, fc2_w, fc1_b, fc2_b, sched)
    out = _run_k4(y, pos0, pos1)
    return out.reshape(b, s, h)
